# baseline (speedup 1.0000x reference)
import functools

import jax
import jax.numpy as jnp
from jax.experimental import pallas as pl

_NB = 8


def _pick_be(E):
    for be in (2560, 2000, 1600, 1280, 1000, 800, 640, 512, 400, 320, 256, 160, 128, 64, 32, 16, 8):
        if E % be == 0:
            return be
    return E


def _edge_body(g_ref, e_ref, w20_ref, b1_ref, we2_ref, b2_ref, watt_ref,
               batt_ref, *rest):
    mpre = (g_ref[...]
            + jnp.dot(e_ref[...], w20_ref[...], preferred_element_type=jnp.float32)
            + b1_ref[...])
    m1 = jax.nn.silu(mpre)
    m = jax.nn.silu(
        jnp.dot(m1, we2_ref[...], preferred_element_type=jnp.float32) + b2_ref[...])
    attl = jnp.sum(m * watt_ref[...], axis=-1, keepdims=True) + batt_ref[:, :1]
    m = m * jax.nn.sigmoid(attl)
    if len(rest) == 6:
        wc1_ref, bc1_ref, wc2_ref, bc2_ref, m_ref, cw_ref = rest
        t = jax.nn.silu(
            jnp.dot(m, wc1_ref[...], preferred_element_type=jnp.float32) + bc1_ref[...])
        cw = jnp.tanh(jnp.sum(t * wc2_ref[...], axis=-1, keepdims=True) + bc2_ref[:, :1])
        m_ref[...] = m
        cw_ref[...] = cw
    else:
        (m_ref,) = rest
        m_ref[...] = m


def _edge_mlp(G, ext, W20, b1, We2, b2, watt, batt, coords=None):
    E, H = G.shape
    BE = _pick_be(E)
    grid = (E // BE,)
    row = lambda v: jnp.reshape(v, (1, -1))
    full = lambda a: pl.BlockSpec(a.shape, lambda i: (0,) * a.ndim)
    ins = [
        pl.BlockSpec((BE, H), lambda i: (i, 0)),
        pl.BlockSpec((BE, ext.shape[1]), lambda i: (i, 0)),
    ]
    args = [G, ext, W20, row(b1), We2, row(b2), row(watt),
            jnp.broadcast_to(jnp.reshape(batt, (1, 1)), (1, H))]
    ins += [full(a) for a in args[2:]]
    if coords is not None:
        Wc1, bc1, wc2, bc2 = coords
        extra = [Wc1, row(bc1), row(wc2),
                 jnp.broadcast_to(jnp.reshape(bc2, (1, 1)), (1, H))]
        args += extra
        ins += [full(a) for a in extra]
        out_shape = (jax.ShapeDtypeStruct((E, H), jnp.float32),
                     jax.ShapeDtypeStruct((E, 1), jnp.float32))
        out_specs = (pl.BlockSpec((BE, H), lambda i: (i, 0)),
                     pl.BlockSpec((BE, 1), lambda i: (i, 0)))
    else:
        out_shape = jax.ShapeDtypeStruct((E, H), jnp.float32)
        out_specs = pl.BlockSpec((BE, H), lambda i: (i, 0))
    return pl.pallas_call(
        _edge_body,
        grid=grid,
        in_specs=ins,
        out_specs=out_specs,
        out_shape=out_shape,
    )(*args)


def _rbf(d, dmin=0.0, dmax=20.0, bins=16):
    centers = jnp.linspace(dmin, dmax, bins)
    width = (dmax - dmin) / bins
    return jnp.exp(-(((d[..., None] - centers) / width) ** 2))


def kernel(X, structure_feat, seq_feat, edge_index, batch_id, params):
    p = params
    egnn = p['egnn']
    n = X.shape[0]
    src = edge_index[0]
    dst = edge_index[1]
    ca = X[:, 1, :]

    pairs = [(0, 1), (0, 2), (0, 3), (1, 2), (1, 3), (2, 3)]
    dists = jnp.stack(
        [jnp.sqrt(jnp.sum((X[:, i] - X[:, j]) ** 2, -1) + 1e-8) for i, j in pairs],
        axis=-1)
    vecs = []
    for i in (0, 2, 3):
        v = X[:, i] - ca
        vecs.append(v / (jnp.sqrt(jnp.sum(v ** 2, -1, keepdims=True)) + 1e-8))
    h_V = jnp.concatenate([dists] + vecs, axis=-1)
    dvec = ca[dst] - ca[src]
    d = jnp.sqrt(jnp.sum(dvec ** 2, -1) + 1e-8)
    h_E = jnp.concatenate([_rbf(d), dvec / (d[:, None] + 1e-8)], axis=-1)

    sfeat = jnp.concatenate([structure_feat, h_V], axis=-1)
    h = sfeat @ egnn['emb_in'][0] + egnn['emb_in'][1]
    x = ca
    ones = jnp.ones((src.shape[0], 1), dtype=h.dtype)
    cnt = jnp.clip(jax.ops.segment_sum(ones, src, n), 1.0)

    for li, lp in enumerate(egnn['layers']):
        We1w, We1b = lp['We1']
        Wsrc, Wdst = We1w[:128], We1w[128:256]
        W20 = We1w[256:]
        Ta = h @ Wsrc
        Tb = h @ Wdst
        G = Ta[src] + Tb[dst]
        cdiff = x[src] - x[dst]
        radial = jnp.sum(cdiff ** 2, -1, keepdims=True)
        ext = jnp.concatenate([radial, h_E], axis=-1)
        if li == 0:
            m, cw = _edge_mlp(
                G, ext, W20, We1b, lp['We2'][0], lp['We2'][1],
                lp['Watt'][0][:, 0], lp['Watt'][1],
                coords=(lp['Wc1'][0], lp['Wc1'][1], lp['Wc2'][0][:, 0], lp['Wc2'][1]))
            cdn = cdiff / (jnp.sqrt(radial) + 1.0)
            x = x + jax.ops.segment_sum(cdn * cw, src, n) / cnt
        else:
            m = _edge_mlp(G, ext, W20, We1b, lp['We2'][0], lp['We2'][1],
                          lp['Watt'][0][:, 0], lp['Watt'][1])
        agg = jax.ops.segment_sum(m, src, n)
        Wn1w, Wn1b = lp['Wn1']
        h_new = jax.nn.silu(h @ Wn1w[:128] + agg @ Wn1w[128:] + Wn1b)
        h = h + h_new @ lp['Wn2'][0] + lp['Wn2'][1]

    node_d1 = h @ egnn['emb_out'][0] + egnn['emb_out'][1]
    seq_d1 = seq_feat @ p['seq'][0] + p['seq'][1]
    emb = jnp.concatenate([node_d1, seq_d1], axis=-1)

    s = jnp.tanh(emb @ p['attn_fc1'][0] + p['attn_fc1'][1])
    s = s @ p['attn_fc2'][0] + p['attn_fc2'][1]
    smax = jax.ops.segment_max(s, batch_id, _NB)
    es = jnp.exp(s - smax[batch_id])
    denom = jax.ops.segment_sum(es, batch_id, _NB)
    att = es / denom[batch_id]
    w = jnp.sum(att, axis=-1)
    pooled = jax.ops.segment_sum(emb * w[:, None], batch_id, _NB)
    emb2 = jax.nn.elu(pooled @ p['proj'][0] + p['proj'][1])
    return emb2 @ p['out'][0] + p['out'][1]


# speedup vs baseline: 1.4458x; 1.4458x over previous
import functools

import jax
import jax.numpy as jnp
from jax import lax
from jax.experimental import pallas as pl
from jax.experimental.pallas import tpu as pltpu
from jax.experimental.pallas import tpu_sc as plsc

_NB = 8
_NC = 2
_NS = 16
_NW = _NC * _NS


def _sc_mesh():
    return plsc.VectorSubcoreMesh(core_axis_name="c", subcore_axis_name="s",
                                  num_cores=_NC, num_subcores=_NS)


def _sc_gather_pair(Tsrc, Tdst, src, dst):
    n, D = Tsrc.shape
    E = src.shape[0]
    per_w = E // _NW
    C = 80
    U = 5
    steps = per_w // (C * U)
    assert per_w == steps * C * U, (E, per_w)

    @functools.partial(
        pl.kernel,
        mesh=_sc_mesh(),
        out_type=(jax.ShapeDtypeStruct((E, D), jnp.float32),
                  jax.ShapeDtypeStruct((E, D), jnp.float32)),
        scratch_types=(
            [pltpu.VMEM((C * U,), jnp.int32) for _ in range(2)]
            + [pltpu.VMEM((C, D), jnp.float32) for _ in range(2 * U)]
            + [pltpu.SemaphoreType.DMA for _ in range(4)]),
    )
    def k(ts_hbm, td_hbm, src_hbm, dst_hbm, ga_hbm, gb_hbm,
          idxa, idxb, *rest):
        bufs = rest[:2 * U]
        s_ia, s_ib, s_g, s_w = rest[2 * U:]
        cid = lax.axis_index("c")
        sid = lax.axis_index("s")
        base = (sid * _NC + cid) * per_w

        def step(i, carry):
            off = base + i * C * U
            ca = pltpu.async_copy(src_hbm.at[pl.ds(off, C * U)], idxa, s_ia)
            cb = pltpu.async_copy(dst_hbm.at[pl.ds(off, C * U)], idxb, s_ib)
            ca.wait()
            cb.wait()
            gs = []
            for u in range(U):
                gs.append(pltpu.async_copy(
                    ts_hbm.at[idxa.at[pl.ds(u * C, C)]], bufs[2 * u], s_g))
                gs.append(pltpu.async_copy(
                    td_hbm.at[idxb.at[pl.ds(u * C, C)]], bufs[2 * u + 1], s_g))
            ws = []
            for u in range(U):
                gs[2 * u].wait()
                ws.append(pltpu.async_copy(
                    bufs[2 * u], ga_hbm.at[pl.ds(off + u * C, C), :], s_w))
                gs[2 * u + 1].wait()
                ws.append(pltpu.async_copy(
                    bufs[2 * u + 1], gb_hbm.at[pl.ds(off + u * C, C), :], s_w))
            for w in ws:
                w.wait()
            return carry

        lax.fori_loop(0, steps, step, 0)

    return k(Tsrc, Tdst, src, dst)


def _sc_scatter_add(vals, idx, n):
    E, D = vals.shape
    per_w = E // _NW
    C = 40
    U = 5
    steps = per_w // (C * U)
    assert per_w == steps * C * U, (E, per_w)
    rows_t = (n // _NS) & ~7
    rows_last = n - rows_t * (_NS - 1)
    zrows = jnp.zeros((rows_last, D), jnp.float32)

    @functools.partial(
        pl.kernel,
        mesh=_sc_mesh(),
        out_type=jax.ShapeDtypeStruct((_NC * n, D), jnp.float32),
        scratch_types=(
            [pltpu.VMEM((C,), jnp.int32) for _ in range(U)]
            + [pltpu.VMEM((C, D), jnp.float32) for _ in range(U)]
            + [pltpu.VMEM_SHARED((n, D), jnp.float32)]
            + [pltpu.SemaphoreType.DMA for _ in range(3)]),
    )
    def k(vals_hbm, idx_hbm, z_hbm, out_hbm, *rest):
        idxs = rest[:U]
        bufs = rest[U:2 * U]
        acc = rest[2 * U]
        s_i, s_v, s_a = rest[2 * U + 1:]
        cid = lax.axis_index("c")
        sid = lax.axis_index("s")
        base = (sid * _NC + cid) * per_w

        @pl.when(sid < _NS - 1)
        def _():
            pltpu.sync_copy(z_hbm.at[pl.ds(0, rows_t), :],
                            acc.at[pl.ds(sid * rows_t, rows_t), :])

        @pl.when(sid == _NS - 1)
        def _():
            pltpu.sync_copy(z_hbm,
                            acc.at[pl.ds((_NS - 1) * rows_t, rows_last), :])

        plsc.subcore_barrier()

        def step(i, carry):
            off = base + i * C * U
            cs = []
            for u in range(U):
                cs.append(pltpu.async_copy(
                    idx_hbm.at[pl.ds(off + u * C, C)], idxs[u], s_i))
                cs.append(pltpu.async_copy(
                    vals_hbm.at[pl.ds(off + u * C, C), :], bufs[u], s_v))
            adds = []
            for u in range(U):
                cs[2 * u].wait()
                cs[2 * u + 1].wait()
                adds.append(pltpu.async_copy(
                    bufs[u], acc.at[idxs[u]], s_a, add=True))
            for a in adds:
                a.wait()
            return carry

        lax.fori_loop(0, steps, step, 0)
        plsc.subcore_barrier()

        @pl.when(sid < _NS - 1)
        def _():
            pltpu.sync_copy(acc.at[pl.ds(sid * rows_t, rows_t), :],
                            out_hbm.at[pl.ds(cid * n + sid * rows_t, rows_t), :])

        @pl.when(sid == _NS - 1)
        def _():
            pltpu.sync_copy(
                acc.at[pl.ds((_NS - 1) * rows_t, rows_last), :],
                out_hbm.at[pl.ds(cid * n + (_NS - 1) * rows_t, rows_last), :])

    out = k(vals, idx, zrows)
    return out[:n] + out[n:]


def _pick_be(E):
    for be in (2560, 2000, 1600, 1280, 1000, 800, 640, 512, 400, 320, 256, 160, 128, 64, 32, 16, 8):
        if E % be == 0:
            return be
    return E


def _edge_body(ga_ref, gb_ref, e_ref, w20_ref, b1_ref, we2_ref, b2_ref, watt_ref,
               batt_ref, *rest):
    mpre = (ga_ref[...] + gb_ref[...]
            + jnp.dot(e_ref[...], w20_ref[...], preferred_element_type=jnp.float32)
            + b1_ref[...])
    m1 = jax.nn.silu(mpre)
    m = jax.nn.silu(
        jnp.dot(m1, we2_ref[...], preferred_element_type=jnp.float32) + b2_ref[...])
    attl = jnp.sum(m * watt_ref[...], axis=-1, keepdims=True) + batt_ref[:, :1]
    m = m * jax.nn.sigmoid(attl)
    if len(rest) == 6:
        wc1_ref, bc1_ref, wc2_ref, bc2_ref, m_ref, cw_ref = rest
        t = jax.nn.silu(
            jnp.dot(m, wc1_ref[...], preferred_element_type=jnp.float32) + bc1_ref[...])
        cw = jnp.tanh(jnp.sum(t * wc2_ref[...], axis=-1, keepdims=True) + bc2_ref[:, :1])
        m_ref[...] = m
        cw_ref[...] = cw
    else:
        (m_ref,) = rest
        m_ref[...] = m


def _edge_mlp(GA, GB, ext, W20, b1, We2, b2, watt, batt, coords=None):
    E, H = GA.shape
    BE = _pick_be(E)
    grid = (E // BE,)
    row = lambda v: jnp.reshape(v, (1, -1))
    full = lambda a: pl.BlockSpec(a.shape, lambda i: (0,) * a.ndim)
    ins = [
        pl.BlockSpec((BE, H), lambda i: (i, 0)),
        pl.BlockSpec((BE, H), lambda i: (i, 0)),
        pl.BlockSpec((BE, ext.shape[1]), lambda i: (i, 0)),
    ]
    args = [GA, GB, ext, W20, row(b1), We2, row(b2), row(watt),
            jnp.broadcast_to(jnp.reshape(batt, (1, 1)), (1, H))]
    ins += [full(a) for a in args[3:]]
    if coords is not None:
        Wc1, bc1, wc2, bc2 = coords
        extra = [Wc1, row(bc1), row(wc2),
                 jnp.broadcast_to(jnp.reshape(bc2, (1, 1)), (1, H))]
        args += extra
        ins += [full(a) for a in extra]
        out_shape = (jax.ShapeDtypeStruct((E, H), jnp.float32),
                     jax.ShapeDtypeStruct((E, 1), jnp.float32))
        out_specs = (pl.BlockSpec((BE, H), lambda i: (i, 0)),
                     pl.BlockSpec((BE, 1), lambda i: (i, 0)))
    else:
        out_shape = jax.ShapeDtypeStruct((E, H), jnp.float32)
        out_specs = pl.BlockSpec((BE, H), lambda i: (i, 0))
    return pl.pallas_call(
        _edge_body,
        grid=grid,
        in_specs=ins,
        out_specs=out_specs,
        out_shape=out_shape,
    )(*args)


def _rbf(d, dmin=0.0, dmax=20.0, bins=16):
    centers = jnp.linspace(dmin, dmax, bins)
    width = (dmax - dmin) / bins
    return jnp.exp(-(((d[..., None] - centers) / width) ** 2))


def kernel(X, structure_feat, seq_feat, edge_index, batch_id, params):
    p = params
    egnn = p['egnn']
    n = X.shape[0]
    src = edge_index[0]
    dst = edge_index[1]
    ca = X[:, 1, :]

    pairs = [(0, 1), (0, 2), (0, 3), (1, 2), (1, 3), (2, 3)]
    dists = jnp.stack(
        [jnp.sqrt(jnp.sum((X[:, i] - X[:, j]) ** 2, -1) + 1e-8) for i, j in pairs],
        axis=-1)
    vecs = []
    for i in (0, 2, 3):
        v = X[:, i] - ca
        vecs.append(v / (jnp.sqrt(jnp.sum(v ** 2, -1, keepdims=True)) + 1e-8))
    h_V = jnp.concatenate([dists] + vecs, axis=-1)
    dvec = ca[dst] - ca[src]
    d = jnp.sqrt(jnp.sum(dvec ** 2, -1) + 1e-8)
    h_E = jnp.concatenate([_rbf(d), dvec / (d[:, None] + 1e-8)], axis=-1)

    sfeat = jnp.concatenate([structure_feat, h_V], axis=-1)
    h = sfeat @ egnn['emb_in'][0] + egnn['emb_in'][1]
    x = ca
    ones = jnp.ones((src.shape[0], 1), dtype=h.dtype)
    cnt = jnp.clip(jax.ops.segment_sum(ones, src, n), 1.0)

    for li, lp in enumerate(egnn['layers']):
        We1w, We1b = lp['We1']
        Wsrc, Wdst = We1w[:128], We1w[128:256]
        W20 = We1w[256:]
        Ta = h @ Wsrc
        Tb = h @ Wdst
        GA, GB = _sc_gather_pair(Ta, Tb, src, dst)
        cdiff = x[src] - x[dst]
        radial = jnp.sum(cdiff ** 2, -1, keepdims=True)
        ext = jnp.concatenate([radial, h_E], axis=-1)
        if li == 0:
            m, cw = _edge_mlp(
                GA, GB, ext, W20, We1b, lp['We2'][0], lp['We2'][1],
                lp['Watt'][0][:, 0], lp['Watt'][1],
                coords=(lp['Wc1'][0], lp['Wc1'][1], lp['Wc2'][0][:, 0], lp['Wc2'][1]))
            cdn = cdiff / (jnp.sqrt(radial) + 1.0)
            x = x + jax.ops.segment_sum(cdn * cw, src, n) / cnt
        else:
            m = _edge_mlp(GA, GB, ext, W20, We1b, lp['We2'][0], lp['We2'][1],
                          lp['Watt'][0][:, 0], lp['Watt'][1])
        agg = _sc_scatter_add(m, src, n)
        Wn1w, Wn1b = lp['Wn1']
        h_new = jax.nn.silu(h @ Wn1w[:128] + agg @ Wn1w[128:] + Wn1b)
        h = h + h_new @ lp['Wn2'][0] + lp['Wn2'][1]

    node_d1 = h @ egnn['emb_out'][0] + egnn['emb_out'][1]
    seq_d1 = seq_feat @ p['seq'][0] + p['seq'][1]
    emb = jnp.concatenate([node_d1, seq_d1], axis=-1)

    s = jnp.tanh(emb @ p['attn_fc1'][0] + p['attn_fc1'][1])
    s = s @ p['attn_fc2'][0] + p['attn_fc2'][1]
    smax = jax.ops.segment_max(s, batch_id, _NB)
    es = jnp.exp(s - smax[batch_id])
    denom = jax.ops.segment_sum(es, batch_id, _NB)
    att = es / denom[batch_id]
    w = jnp.sum(att, axis=-1)
    pooled = jax.ops.segment_sum(emb * w[:, None], batch_id, _NB)
    emb2 = jax.nn.elu(pooled @ p['proj'][0] + p['proj'][1])
    return emb2 @ p['out'][0] + p['out'][1]


# speedup vs baseline: 3.8117x; 2.6364x over previous
import functools

import jax
import jax.numpy as jnp
from jax import lax
from jax.experimental import pallas as pl
from jax.experimental.pallas import tpu as pltpu
from jax.experimental.pallas import tpu_sc as plsc

_NB = 8
_NC = 2
_NS = 16
_NW = _NC * _NS


def _sc_mesh():
    return plsc.VectorSubcoreMesh(core_axis_name="c", subcore_axis_name="s",
                                  num_cores=_NC, num_subcores=_NS)


def _sc_gather_pair(Tsrc, Tdst, src, dst):
    n, D = Tsrc.shape
    E = src.shape[0]
    per_w = E // _NW
    C = 40 if D > 128 else 80
    U = 5
    steps = per_w // (C * U)
    assert per_w == steps * C * U, (E, per_w)

    @functools.partial(
        pl.kernel,
        mesh=_sc_mesh(),
        out_type=(jax.ShapeDtypeStruct((E, D), jnp.float32),
                  jax.ShapeDtypeStruct((E, D), jnp.float32)),
        scratch_types=(
            [pltpu.VMEM((C * U,), jnp.int32) for _ in range(2)]
            + [pltpu.VMEM((C, D), jnp.float32) for _ in range(2 * U)]
            + [pltpu.SemaphoreType.DMA for _ in range(4)]),
    )
    def k(ts_hbm, td_hbm, src_hbm, dst_hbm, ga_hbm, gb_hbm,
          idxa, idxb, *rest):
        bufs = rest[:2 * U]
        s_ia, s_ib, s_g, s_w = rest[2 * U:]
        cid = lax.axis_index("c")
        sid = lax.axis_index("s")
        base = (sid * _NC + cid) * per_w

        def step(i, carry):
            off = base + i * C * U
            ca = pltpu.async_copy(src_hbm.at[pl.ds(off, C * U)], idxa, s_ia)
            cb = pltpu.async_copy(dst_hbm.at[pl.ds(off, C * U)], idxb, s_ib)
            ca.wait()
            cb.wait()
            gs = []
            for u in range(U):
                gs.append(pltpu.async_copy(
                    ts_hbm.at[idxa.at[pl.ds(u * C, C)]], bufs[2 * u], s_g))
                gs.append(pltpu.async_copy(
                    td_hbm.at[idxb.at[pl.ds(u * C, C)]], bufs[2 * u + 1], s_g))
            ws = []
            for u in range(U):
                gs[2 * u].wait()
                ws.append(pltpu.async_copy(
                    bufs[2 * u], ga_hbm.at[pl.ds(off + u * C, C), :], s_w))
                gs[2 * u + 1].wait()
                ws.append(pltpu.async_copy(
                    bufs[2 * u + 1], gb_hbm.at[pl.ds(off + u * C, C), :], s_w))
            for w in ws:
                w.wait()
            return carry

        lax.fori_loop(0, steps, step, 0)

    return k(Tsrc, Tdst, src, dst)


def _sc_scatter_add(vals, idx, n, vals2=None):
    E, D = vals.shape
    D2 = 0 if vals2 is None else vals2.shape[1]
    per_w = E // _NW
    C = 40
    U = 5
    steps = per_w // (C * U)
    assert per_w == steps * C * U, (E, per_w)
    rows_t = (n // _NS) & ~7
    rows_last = n - rows_t * (_NS - 1)
    zrows = jnp.zeros((rows_last, D), jnp.float32)

    out_type = [jax.ShapeDtypeStruct((_NC * n, D), jnp.float32)]
    scratch = ([pltpu.VMEM((C,), jnp.int32) for _ in range(U)]
               + [pltpu.VMEM((C, D), jnp.float32) for _ in range(U)]
               + [pltpu.VMEM_SHARED((n, D), jnp.float32)]
               + [pltpu.SemaphoreType.DMA for _ in range(3)])
    ins = [vals, idx, zrows]
    if vals2 is not None:
        out_type.append(jax.ShapeDtypeStruct((_NC * n, D2), jnp.float32))
        scratch += ([pltpu.VMEM((C, D2), jnp.float32) for _ in range(U)]
                    + [pltpu.VMEM_SHARED((n, D2), jnp.float32)])
        ins += [vals2, jnp.zeros((rows_last, D2), jnp.float32)]
    nin = len(ins)
    nout = len(out_type)

    @functools.partial(pl.kernel, mesh=_sc_mesh(), out_type=tuple(out_type),
                       scratch_types=tuple(scratch))
    def k(*refs):
        if vals2 is None:
            vals_hbm, idx_hbm, z_hbm, out_hbm = refs[:4]
        else:
            (vals_hbm, idx_hbm, z_hbm, v2_hbm, z2_hbm,
             out_hbm, out2_hbm) = refs[:7]
        rest = refs[nin + nout:]
        idxs = rest[:U]
        bufs = rest[U:2 * U]
        acc = rest[2 * U]
        s_i, s_v, s_a = rest[2 * U + 1:2 * U + 4]
        if vals2 is not None:
            bufs2 = rest[2 * U + 4:3 * U + 4]
            acc2 = rest[3 * U + 4]
        cid = lax.axis_index("c")
        sid = lax.axis_index("s")
        base = (sid * _NC + cid) * per_w

        def zero_wb(spm, hbm, writeback):
            if writeback:
                lo = pl.ds(sid * rows_t, rows_t)
                lo_o = pl.ds(cid * n + sid * rows_t, rows_t)
                hi = pl.ds((_NS - 1) * rows_t, rows_last)
                hi_o = pl.ds(cid * n + (_NS - 1) * rows_t, rows_last)

                @pl.when(sid < _NS - 1)
                def _():
                    pltpu.sync_copy(spm.at[lo, :], hbm.at[lo_o, :])

                @pl.when(sid == _NS - 1)
                def _():
                    pltpu.sync_copy(spm.at[hi, :], hbm.at[hi_o, :])
            else:

                @pl.when(sid < _NS - 1)
                def _():
                    pltpu.sync_copy(hbm.at[pl.ds(0, rows_t), :],
                                    spm.at[pl.ds(sid * rows_t, rows_t), :])

                @pl.when(sid == _NS - 1)
                def _():
                    pltpu.sync_copy(
                        hbm,
                        spm.at[pl.ds((_NS - 1) * rows_t, rows_last), :])

        zero_wb(acc, z_hbm, False)
        if vals2 is not None:
            zero_wb(acc2, z2_hbm, False)
        plsc.subcore_barrier()

        def step(i, carry):
            off = base + i * C * U
            cs = []
            for u in range(U):
                cs.append(pltpu.async_copy(
                    idx_hbm.at[pl.ds(off + u * C, C)], idxs[u], s_i))
                cs.append(pltpu.async_copy(
                    vals_hbm.at[pl.ds(off + u * C, C), :], bufs[u], s_v))
                if vals2 is not None:
                    cs.append(pltpu.async_copy(
                        v2_hbm.at[pl.ds(off + u * C, C), :], bufs2[u], s_v))
            adds = []
            k = 3 if vals2 is not None else 2
            for u in range(U):
                for j in range(k):
                    cs[k * u + j].wait()
                adds.append(pltpu.async_copy(
                    bufs[u], acc.at[idxs[u]], s_a, add=True))
                if vals2 is not None:
                    adds.append(pltpu.async_copy(
                        bufs2[u], acc2.at[idxs[u]], s_a, add=True))
            for a in adds:
                a.wait()
            return carry

        lax.fori_loop(0, steps, step, 0)
        plsc.subcore_barrier()
        zero_wb(acc, out_hbm, True)
        if vals2 is not None:
            zero_wb(acc2, out2_hbm, True)

    if vals2 is None:
        (out,) = k(*ins)
        return out[:n] + out[n:]
    out, out2 = k(*ins)
    return out[:n] + out[n:], out2[:n] + out2[n:]


def _pick_be(E):
    for be in (2560, 2000, 1600, 1280, 1000, 800, 640, 512, 400, 320, 256, 160, 128, 64, 32, 16, 8):
        if E % be == 0:
            return be
    return E


def _edge_body(ga_ref, gb_ref, wrad_ref, wrbf_ref, wdv_ref,
               b1_ref, we2_ref, b2_ref, watt_ref, batt_ref, *rest):
    ga = ga_ref[...]
    gb = gb_ref[...]
    cdiff = ga[:, 128:131] - gb[:, 128:131]
    dvec = gb[:, 131:134] - ga[:, 131:134]
    radial = jnp.sum(cdiff * cdiff, axis=-1, keepdims=True)
    d = jnp.sqrt(jnp.sum(dvec * dvec, axis=-1, keepdims=True) + 1e-8)
    dvn = dvec / (d + 1e-8)
    centers = (lax.broadcasted_iota(jnp.int32, (1, 16), 1).astype(jnp.float32)
               * (20.0 / 15.0))
    rbf = jnp.exp(-(((d - centers) / 1.25) ** 2))
    mpre = (ga[:, 0:128] + gb[:, 0:128]
            + radial * wrad_ref[...]
            + jnp.dot(rbf, wrbf_ref[...], preferred_element_type=jnp.float32)
            + jnp.dot(dvn, wdv_ref[...], preferred_element_type=jnp.float32)
            + b1_ref[...])
    m1 = jax.nn.silu(mpre)
    m = jax.nn.silu(
        jnp.dot(m1, we2_ref[...], preferred_element_type=jnp.float32) + b2_ref[...])
    attl = jnp.sum(m * watt_ref[...], axis=-1, keepdims=True) + batt_ref[:, :1]
    m = m * jax.nn.sigmoid(attl)
    if len(rest) == 6:
        wc1_ref, bc1_ref, wc2_ref, bc2_ref, m_ref, c8_ref = rest
        t = jax.nn.silu(
            jnp.dot(m, wc1_ref[...], preferred_element_type=jnp.float32) + bc1_ref[...])
        cw = jnp.tanh(jnp.sum(t * wc2_ref[...], axis=-1, keepdims=True) + bc2_ref[:, :1])
        cdncw = (cdiff / (jnp.sqrt(radial) + 1.0)) * cw
        m_ref[...] = m
        c8_ref[...] = jnp.concatenate(
            [cdncw, jnp.zeros_like(cdncw),
             jnp.ones((cdncw.shape[0], 1), jnp.float32),
             jnp.zeros((cdncw.shape[0], 121), jnp.float32)], axis=-1)
    else:
        (m_ref,) = rest
        m_ref[...] = m


def _edge_mlp(GA, GB, W20, b1, We2, b2, watt, batt, coords=None):
    E, W = GA.shape
    H = 128
    BE = _pick_be(E)
    grid = (E // BE,)
    row = lambda v: jnp.reshape(v, (1, -1))
    full = lambda a: pl.BlockSpec(a.shape, lambda i: (0,) * a.ndim)
    ins = [
        pl.BlockSpec((BE, W), lambda i: (i, 0)),
        pl.BlockSpec((BE, W), lambda i: (i, 0)),
    ]
    args = [GA, GB, row(W20[0]), W20[1:17], W20[17:20],
            row(b1), We2, row(b2), row(watt),
            jnp.broadcast_to(jnp.reshape(batt, (1, 1)), (1, H))]
    ins += [full(a) for a in args[2:]]
    if coords is not None:
        Wc1, bc1, wc2, bc2 = coords
        extra = [Wc1, row(bc1), row(wc2),
                 jnp.broadcast_to(jnp.reshape(bc2, (1, 1)), (1, H))]
        args += extra
        ins += [full(a) for a in extra]
        out_shape = (jax.ShapeDtypeStruct((E, H), jnp.float32),
                     jax.ShapeDtypeStruct((E, H), jnp.float32))
        out_specs = (pl.BlockSpec((BE, H), lambda i: (i, 0)),
                     pl.BlockSpec((BE, H), lambda i: (i, 0)))
    else:
        out_shape = jax.ShapeDtypeStruct((E, H), jnp.float32)
        out_specs = pl.BlockSpec((BE, H), lambda i: (i, 0))
    return pl.pallas_call(
        _edge_body,
        grid=grid,
        in_specs=ins,
        out_specs=out_specs,
        out_shape=out_shape,
    )(*args)


def kernel(X, structure_feat, seq_feat, edge_index, batch_id, params):
    p = params
    egnn = p['egnn']
    n = X.shape[0]
    src = edge_index[0]
    dst = edge_index[1]
    ca = X[:, 1, :]

    pairs = [(0, 1), (0, 2), (0, 3), (1, 2), (1, 3), (2, 3)]
    dists = jnp.stack(
        [jnp.sqrt(jnp.sum((X[:, i] - X[:, j]) ** 2, -1) + 1e-8) for i, j in pairs],
        axis=-1)
    vecs = []
    for i in (0, 2, 3):
        v = X[:, i] - ca
        vecs.append(v / (jnp.sqrt(jnp.sum(v ** 2, -1, keepdims=True)) + 1e-8))
    h_V = jnp.concatenate([dists] + vecs, axis=-1)

    sfeat = jnp.concatenate([structure_feat, h_V], axis=-1)
    h = sfeat @ egnn['emb_in'][0] + egnn['emb_in'][1]
    x = ca

    for li, lp in enumerate(egnn['layers']):
        We1w, We1b = lp['We1']
        Wsrc, Wdst = We1w[:128], We1w[128:256]
        W20 = We1w[256:]
        pad = jnp.zeros((n, 122), jnp.float32)
        Tsrc = jnp.concatenate([h @ Wsrc, x, ca, pad], axis=-1)
        Tdst = jnp.concatenate([h @ Wdst, x, ca, pad], axis=-1)
        GA, GB = _sc_gather_pair(Tsrc, Tdst, src, dst)
        if li == 0:
            m, c8 = _edge_mlp(
                GA, GB, W20, We1b, lp['We2'][0], lp['We2'][1],
                lp['Watt'][0][:, 0], lp['Watt'][1],
                coords=(lp['Wc1'][0], lp['Wc1'][1], lp['Wc2'][0][:, 0], lp['Wc2'][1]))
            agg = _sc_scatter_add(m, src, n)
            c8agg = _sc_scatter_add(c8, src, n)
            cnt = jnp.clip(c8agg[:, 6:7], 1.0)
            x = x + c8agg[:, :3] / cnt
        else:
            m = _edge_mlp(GA, GB, W20, We1b, lp['We2'][0], lp['We2'][1],
                          lp['Watt'][0][:, 0], lp['Watt'][1])
            agg = _sc_scatter_add(m, src, n)
        Wn1w, Wn1b = lp['Wn1']
        h_new = jax.nn.silu(h @ Wn1w[:128] + agg @ Wn1w[128:] + Wn1b)
        h = h + h_new @ lp['Wn2'][0] + lp['Wn2'][1]

    node_d1 = h @ egnn['emb_out'][0] + egnn['emb_out'][1]
    seq_d1 = seq_feat @ p['seq'][0] + p['seq'][1]
    emb = jnp.concatenate([node_d1, seq_d1], axis=-1)

    s = jnp.tanh(emb @ p['attn_fc1'][0] + p['attn_fc1'][1])
    s = s @ p['attn_fc2'][0] + p['attn_fc2'][1]
    smax = jax.ops.segment_max(s, batch_id, _NB)
    es = jnp.exp(s - smax[batch_id])
    denom = jax.ops.segment_sum(es, batch_id, _NB)
    att = es / denom[batch_id]
    w = jnp.sum(att, axis=-1)
    pooled = jax.ops.segment_sum(emb * w[:, None], batch_id, _NB)
    emb2 = jax.nn.elu(pooled @ p['proj'][0] + p['proj'][1])
    return emb2 @ p['out'][0] + p['out'][1]


# speedup vs baseline: 4.4019x; 1.1549x over previous
import functools

import jax
import jax.numpy as jnp
from jax import lax
from jax.experimental import pallas as pl
from jax.experimental.pallas import tpu as pltpu
from jax.experimental.pallas import tpu_sc as plsc

_NB = 8
_NC = 2
_NS = 16
_NW = _NC * _NS


def _sc_mesh():
    return plsc.VectorSubcoreMesh(core_axis_name="c", subcore_axis_name="s",
                                  num_cores=_NC, num_subcores=_NS)


def _sc_gather_pair(Tsrc, Tdst, src, dst):
    n, D = Tsrc.shape
    E = src.shape[0]
    dt = Tsrc.dtype
    per_w = E // _NW
    itemsize = 2 if dt == jnp.bfloat16 else 4
    C = 80 if D * itemsize <= 512 else 40
    U = 5
    steps = per_w // (C * U)
    assert per_w == steps * C * U, (E, per_w)

    @functools.partial(
        pl.kernel,
        mesh=_sc_mesh(),
        out_type=(jax.ShapeDtypeStruct((E, D), dt),
                  jax.ShapeDtypeStruct((E, D), dt)),
        scratch_types=(
            [pltpu.VMEM((C * U,), jnp.int32) for _ in range(2)]
            + [pltpu.VMEM((C, D), dt) for _ in range(2 * U)]
            + [pltpu.SemaphoreType.DMA for _ in range(4)]),
    )
    def k(ts_hbm, td_hbm, src_hbm, dst_hbm, ga_hbm, gb_hbm,
          idxa, idxb, *rest):
        bufs = rest[:2 * U]
        s_ia, s_ib, s_g, s_w = rest[2 * U:]
        cid = lax.axis_index("c")
        sid = lax.axis_index("s")
        base = (sid * _NC + cid) * per_w

        def step(i, carry):
            off = base + i * C * U
            ca = pltpu.async_copy(src_hbm.at[pl.ds(off, C * U)], idxa, s_ia)
            cb = pltpu.async_copy(dst_hbm.at[pl.ds(off, C * U)], idxb, s_ib)
            ca.wait()
            cb.wait()
            gs = []
            for u in range(U):
                gs.append(pltpu.async_copy(
                    ts_hbm.at[idxa.at[pl.ds(u * C, C)]], bufs[2 * u], s_g))
                gs.append(pltpu.async_copy(
                    td_hbm.at[idxb.at[pl.ds(u * C, C)]], bufs[2 * u + 1], s_g))
            ws = []
            for u in range(U):
                gs[2 * u].wait()
                ws.append(pltpu.async_copy(
                    bufs[2 * u], ga_hbm.at[pl.ds(off + u * C, C), :], s_w))
                gs[2 * u + 1].wait()
                ws.append(pltpu.async_copy(
                    bufs[2 * u + 1], gb_hbm.at[pl.ds(off + u * C, C), :], s_w))
            for w in ws:
                w.wait()
            return carry

        lax.fori_loop(0, steps, step, 0)

    return k(Tsrc, Tdst, src, dst)


def _sc_scatter_add(vals, idx, n, vals2=None):
    E, D = vals.shape
    D2 = 0 if vals2 is None else vals2.shape[1]
    per_w = E // _NW
    C = 40
    U = 5
    steps = per_w // (C * U)
    assert per_w == steps * C * U, (E, per_w)
    rows_t = (n // _NS) & ~7
    rows_last = n - rows_t * (_NS - 1)
    zrows = jnp.zeros((rows_last, D), jnp.float32)

    out_type = [jax.ShapeDtypeStruct((_NC * n, D), jnp.float32)]
    scratch = ([pltpu.VMEM((C,), jnp.int32) for _ in range(U)]
               + [pltpu.VMEM((C, D), jnp.float32) for _ in range(U)]
               + [pltpu.VMEM_SHARED((n, D), jnp.float32)]
               + [pltpu.SemaphoreType.DMA for _ in range(3)])
    ins = [vals, idx, zrows]
    if vals2 is not None:
        out_type.append(jax.ShapeDtypeStruct((_NC * n, D2), jnp.float32))
        scratch += ([pltpu.VMEM((C, D2), jnp.float32) for _ in range(U)]
                    + [pltpu.VMEM_SHARED((n, D2), jnp.float32)])
        ins += [vals2, jnp.zeros((rows_last, D2), jnp.float32)]
    nin = len(ins)
    nout = len(out_type)

    @functools.partial(pl.kernel, mesh=_sc_mesh(), out_type=tuple(out_type),
                       scratch_types=tuple(scratch))
    def k(*refs):
        if vals2 is None:
            vals_hbm, idx_hbm, z_hbm, out_hbm = refs[:4]
        else:
            (vals_hbm, idx_hbm, z_hbm, v2_hbm, z2_hbm,
             out_hbm, out2_hbm) = refs[:7]
        rest = refs[nin + nout:]
        idxs = rest[:U]
        bufs = rest[U:2 * U]
        acc = rest[2 * U]
        s_i, s_v, s_a = rest[2 * U + 1:2 * U + 4]
        if vals2 is not None:
            bufs2 = rest[2 * U + 4:3 * U + 4]
            acc2 = rest[3 * U + 4]
        cid = lax.axis_index("c")
        sid = lax.axis_index("s")
        base = (sid * _NC + cid) * per_w

        def zero_wb(spm, hbm, writeback):
            if writeback:
                lo = pl.ds(sid * rows_t, rows_t)
                lo_o = pl.ds(cid * n + sid * rows_t, rows_t)
                hi = pl.ds((_NS - 1) * rows_t, rows_last)
                hi_o = pl.ds(cid * n + (_NS - 1) * rows_t, rows_last)

                @pl.when(sid < _NS - 1)
                def _():
                    pltpu.sync_copy(spm.at[lo, :], hbm.at[lo_o, :])

                @pl.when(sid == _NS - 1)
                def _():
                    pltpu.sync_copy(spm.at[hi, :], hbm.at[hi_o, :])
            else:

                @pl.when(sid < _NS - 1)
                def _():
                    pltpu.sync_copy(hbm.at[pl.ds(0, rows_t), :],
                                    spm.at[pl.ds(sid * rows_t, rows_t), :])

                @pl.when(sid == _NS - 1)
                def _():
                    pltpu.sync_copy(
                        hbm,
                        spm.at[pl.ds((_NS - 1) * rows_t, rows_last), :])

        zero_wb(acc, z_hbm, False)
        if vals2 is not None:
            zero_wb(acc2, z2_hbm, False)
        plsc.subcore_barrier()

        def step(i, carry):
            off = base + i * C * U
            cs = []
            for u in range(U):
                cs.append(pltpu.async_copy(
                    idx_hbm.at[pl.ds(off + u * C, C)], idxs[u], s_i))
                cs.append(pltpu.async_copy(
                    vals_hbm.at[pl.ds(off + u * C, C), :], bufs[u], s_v))
                if vals2 is not None:
                    cs.append(pltpu.async_copy(
                        v2_hbm.at[pl.ds(off + u * C, C), :], bufs2[u], s_v))
            adds = []
            k = 3 if vals2 is not None else 2
            for u in range(U):
                for j in range(k):
                    cs[k * u + j].wait()
                adds.append(pltpu.async_copy(
                    bufs[u], acc.at[idxs[u]], s_a, add=True))
                if vals2 is not None:
                    adds.append(pltpu.async_copy(
                        bufs2[u], acc2.at[idxs[u]], s_a, add=True))
            for a in adds:
                a.wait()
            return carry

        lax.fori_loop(0, steps, step, 0)
        plsc.subcore_barrier()
        zero_wb(acc, out_hbm, True)
        if vals2 is not None:
            zero_wb(acc2, out2_hbm, True)

    if vals2 is None:
        (out,) = k(*ins)
        return out[:n] + out[n:]
    out, out2 = k(*ins)
    return out[:n] + out[n:], out2[:n] + out2[n:]


def _pick_be(E):
    for be in (2560, 2000, 1600, 1280, 1000, 800, 640, 512, 400, 320, 256, 160, 128, 64, 32, 16, 8):
        if E % be == 0:
            return be
    return E


def _edge_body(ga_ref, gb_ref, wrad_ref, wrbf_ref, wdv_ref,
               b1_ref, we2_ref, b2_ref, watt_ref, batt_ref, *rest):
    ga = ga_ref[...]
    gb = gb_ref[...]
    fa = lax.bitcast_convert_type(ga << 16, jnp.float32)
    fb = lax.bitcast_convert_type(gb << 16, jnp.float32)
    qa = lax.bitcast_convert_type(ga & jnp.int32(-65536), jnp.float32)
    qb = lax.bitcast_convert_type(gb & jnp.int32(-65536), jnp.float32)
    xs = qa[:, 0:3] + qa[:, 6:9]
    xd = qb[:, 0:3] + qb[:, 6:9]
    cas = qa[:, 3:6] + qa[:, 9:12]
    cad = qb[:, 3:6] + qb[:, 9:12]
    cdiff = xs - xd
    dvec = cad - cas
    radial = jnp.sum(cdiff * cdiff, axis=-1, keepdims=True)
    d = jnp.sqrt(jnp.sum(dvec * dvec, axis=-1, keepdims=True) + 1e-8)
    dvn = dvec / (d + 1e-8)
    centers = (lax.broadcasted_iota(jnp.int32, (1, 16), 1).astype(jnp.float32)
               * (20.0 / 15.0))
    rbf = jnp.exp(-(((d - centers) / 1.25) ** 2))
    mpre = (fa + fb
            + radial * wrad_ref[...]
            + jnp.dot(rbf, wrbf_ref[...], preferred_element_type=jnp.float32)
            + jnp.dot(dvn, wdv_ref[...], preferred_element_type=jnp.float32)
            + b1_ref[...])
    m1 = jax.nn.silu(mpre)
    m = jax.nn.silu(
        jnp.dot(m1, we2_ref[...], preferred_element_type=jnp.float32) + b2_ref[...])
    attl = jnp.sum(m * watt_ref[...], axis=-1, keepdims=True) + batt_ref[:, :1]
    m = m * jax.nn.sigmoid(attl)
    if len(rest) == 6:
        wc1_ref, bc1_ref, wc2_ref, bc2_ref, m_ref, c8_ref = rest
        t = jax.nn.silu(
            jnp.dot(m, wc1_ref[...], preferred_element_type=jnp.float32) + bc1_ref[...])
        cw = jnp.tanh(jnp.sum(t * wc2_ref[...], axis=-1, keepdims=True) + bc2_ref[:, :1])
        cdncw = (cdiff / (jnp.sqrt(radial) + 1.0)) * cw
        m_ref[...] = m
        c8_ref[...] = jnp.concatenate(
            [cdncw, jnp.zeros_like(cdncw),
             jnp.ones((cdncw.shape[0], 1), jnp.float32),
             jnp.zeros((cdncw.shape[0], 121), jnp.float32)], axis=-1)
    else:
        (m_ref,) = rest
        m_ref[...] = m


def _edge_mlp(GA, GB, W20, b1, We2, b2, watt, batt, coords=None):
    E, W = GA.shape
    H = 128
    BE = _pick_be(E)
    grid = (E // BE,)
    row = lambda v: jnp.reshape(v, (1, -1))
    full = lambda a: pl.BlockSpec(a.shape, lambda i: (0,) * a.ndim)
    ins = [
        pl.BlockSpec((BE, W), lambda i: (i, 0)),
        pl.BlockSpec((BE, W), lambda i: (i, 0)),
    ]
    args = [GA, GB, row(W20[0]), W20[1:17], W20[17:20],
            row(b1), We2, row(b2), row(watt),
            jnp.broadcast_to(jnp.reshape(batt, (1, 1)), (1, H))]
    ins += [full(a) for a in args[2:]]
    if coords is not None:
        Wc1, bc1, wc2, bc2 = coords
        extra = [Wc1, row(bc1), row(wc2),
                 jnp.broadcast_to(jnp.reshape(bc2, (1, 1)), (1, H))]
        args += extra
        ins += [full(a) for a in extra]
        out_shape = (jax.ShapeDtypeStruct((E, H), jnp.float32),
                     jax.ShapeDtypeStruct((E, H), jnp.float32))
        out_specs = (pl.BlockSpec((BE, H), lambda i: (i, 0)),
                     pl.BlockSpec((BE, H), lambda i: (i, 0)))
    else:
        out_shape = jax.ShapeDtypeStruct((E, H), jnp.float32)
        out_specs = pl.BlockSpec((BE, H), lambda i: (i, 0))
    return pl.pallas_call(
        _edge_body,
        grid=grid,
        in_specs=ins,
        out_specs=out_specs,
        out_shape=out_shape,
    )(*args)


def kernel(X, structure_feat, seq_feat, edge_index, batch_id, params):
    p = params
    egnn = p['egnn']
    n = X.shape[0]
    src = edge_index[0]
    dst = edge_index[1]
    ca = X[:, 1, :]

    pairs = [(0, 1), (0, 2), (0, 3), (1, 2), (1, 3), (2, 3)]
    dists = jnp.stack(
        [jnp.sqrt(jnp.sum((X[:, i] - X[:, j]) ** 2, -1) + 1e-8) for i, j in pairs],
        axis=-1)
    vecs = []
    for i in (0, 2, 3):
        v = X[:, i] - ca
        vecs.append(v / (jnp.sqrt(jnp.sum(v ** 2, -1, keepdims=True)) + 1e-8))
    h_V = jnp.concatenate([dists] + vecs, axis=-1)

    sfeat = jnp.concatenate([structure_feat, h_V], axis=-1)
    h = sfeat @ egnn['emb_in'][0] + egnn['emb_in'][1]
    x = ca

    for li, lp in enumerate(egnn['layers']):
        We1w, We1b = lp['We1']
        Wsrc, Wdst = We1w[:128], We1w[128:256]
        W20 = We1w[256:]
        bf = jnp.bfloat16
        u16b = lambda v: lax.bitcast_convert_type(v.astype(bf), jnp.uint16)
        x_hi = x.astype(bf)
        x_lo = x - x_hi.astype(jnp.float32)
        ca_hi = ca.astype(bf)
        ca_lo = ca - ca_hi.astype(jnp.float32)
        Q = jnp.concatenate([x, ca, x_lo, ca_lo,
                             jnp.zeros((n, 116), jnp.float32)], axis=-1)
        qw = u16b(Q).astype(jnp.uint32) << 16
        pack = lambda P: lax.bitcast_convert_type(
            qw | u16b(P).astype(jnp.uint32), jnp.int32)
        Tsrc = pack(h @ Wsrc)
        Tdst = pack(h @ Wdst)
        GA, GB = _sc_gather_pair(Tsrc, Tdst, src, dst)
        if li == 0:
            m, c8 = _edge_mlp(
                GA, GB, W20, We1b, lp['We2'][0], lp['We2'][1],
                lp['Watt'][0][:, 0], lp['Watt'][1],
                coords=(lp['Wc1'][0], lp['Wc1'][1], lp['Wc2'][0][:, 0], lp['Wc2'][1]))
            agg = _sc_scatter_add(m, src, n)
            c8agg = _sc_scatter_add(c8, src, n)
            cnt = jnp.clip(c8agg[:, 6:7], 1.0)
            x = x + c8agg[:, :3] / cnt
        else:
            m = _edge_mlp(GA, GB, W20, We1b, lp['We2'][0], lp['We2'][1],
                          lp['Watt'][0][:, 0], lp['Watt'][1])
            agg = _sc_scatter_add(m, src, n)
        Wn1w, Wn1b = lp['Wn1']
        h_new = jax.nn.silu(h @ Wn1w[:128] + agg @ Wn1w[128:] + Wn1b)
        h = h + h_new @ lp['Wn2'][0] + lp['Wn2'][1]

    node_d1 = h @ egnn['emb_out'][0] + egnn['emb_out'][1]
    seq_d1 = seq_feat @ p['seq'][0] + p['seq'][1]
    emb = jnp.concatenate([node_d1, seq_d1], axis=-1)

    s = jnp.tanh(emb @ p['attn_fc1'][0] + p['attn_fc1'][1])
    s = s @ p['attn_fc2'][0] + p['attn_fc2'][1]
    smax = jax.ops.segment_max(s, batch_id, _NB)
    es = jnp.exp(s - smax[batch_id])
    denom = jax.ops.segment_sum(es, batch_id, _NB)
    att = es / denom[batch_id]
    w = jnp.sum(att, axis=-1)
    pooled = jax.ops.segment_sum(emb * w[:, None], batch_id, _NB)
    emb2 = jax.nn.elu(pooled @ p['proj'][0] + p['proj'][1])
    return emb2 @ p['out'][0] + p['out'][1]


# speedup vs baseline: 5.0294x; 1.1425x over previous
import functools

import jax
import jax.numpy as jnp
from jax import lax
from jax.experimental import pallas as pl
from jax.experimental.pallas import tpu as pltpu
from jax.experimental.pallas import tpu_sc as plsc

_NB = 8
_NC = 2
_NS = 16
_NW = _NC * _NS


def _sc_mesh():
    return plsc.VectorSubcoreMesh(core_axis_name="c", subcore_axis_name="s",
                                  num_cores=_NC, num_subcores=_NS)


def _sc_gather_pair(Tsrc, Tdst, src, dst):
    n, D = Tsrc.shape
    E = src.shape[0]
    dt = Tsrc.dtype
    per_w = E // _NW
    itemsize = 2 if dt == jnp.bfloat16 else 4
    C = 80 if D * itemsize <= 512 else 40
    U = 5
    steps = per_w // (C * U)
    assert per_w == steps * C * U, (E, per_w)

    @functools.partial(
        pl.kernel,
        mesh=_sc_mesh(),
        out_type=(jax.ShapeDtypeStruct((E, D), dt),
                  jax.ShapeDtypeStruct((E, D), dt)),
        scratch_types=(
            [pltpu.VMEM((C * U,), jnp.int32) for _ in range(2)]
            + [pltpu.VMEM((C, D), dt) for _ in range(2 * U)]
            + [pltpu.SemaphoreType.DMA for _ in range(4)]),
    )
    def k(ts_hbm, td_hbm, src_hbm, dst_hbm, ga_hbm, gb_hbm,
          idxa, idxb, *rest):
        bufs = rest[:2 * U]
        s_ia, s_ib, s_g, s_w = rest[2 * U:]
        cid = lax.axis_index("c")
        sid = lax.axis_index("s")
        base = (sid * _NC + cid) * per_w

        def step(i, carry):
            off = base + i * C * U
            ca = pltpu.async_copy(src_hbm.at[pl.ds(off, C * U)], idxa, s_ia)
            cb = pltpu.async_copy(dst_hbm.at[pl.ds(off, C * U)], idxb, s_ib)
            ca.wait()
            cb.wait()
            gs = []
            for u in range(U):
                gs.append(pltpu.async_copy(
                    ts_hbm.at[idxa.at[pl.ds(u * C, C)]], bufs[2 * u], s_g))
                gs.append(pltpu.async_copy(
                    td_hbm.at[idxb.at[pl.ds(u * C, C)]], bufs[2 * u + 1], s_g))
            ws = []
            for u in range(U):
                gs[2 * u].wait()
                ws.append(pltpu.async_copy(
                    bufs[2 * u], ga_hbm.at[pl.ds(off + u * C, C), :], s_w))
                gs[2 * u + 1].wait()
                ws.append(pltpu.async_copy(
                    bufs[2 * u + 1], gb_hbm.at[pl.ds(off + u * C, C), :], s_w))
            for w in ws:
                w.wait()
            return carry

        lax.fori_loop(0, steps, step, 0)

    return k(Tsrc, Tdst, src, dst)


def _sc_scatter_add(vals, idx, n, vals2=None):
    E, D = vals.shape
    D2 = 0 if vals2 is None else vals2.shape[1]
    per_w = E // _NW
    C = 40
    U = 5
    steps = per_w // (C * U)
    assert per_w == steps * C * U, (E, per_w)
    rows_t = (n // _NS) & ~7
    rows_last = n - rows_t * (_NS - 1)
    zrows = jnp.zeros((rows_last, D), jnp.float32)

    out_type = [jax.ShapeDtypeStruct((_NC * n, D), jnp.float32)]
    scratch = ([pltpu.VMEM((C,), jnp.int32) for _ in range(U)]
               + [pltpu.VMEM((C, D), jnp.float32) for _ in range(U)]
               + [pltpu.VMEM_SHARED((n, D), jnp.float32)]
               + [pltpu.SemaphoreType.DMA for _ in range(3)])
    ins = [vals, idx, zrows]
    if vals2 is not None:
        out_type.append(jax.ShapeDtypeStruct((_NC * n, D2), jnp.float32))
        scratch += ([pltpu.VMEM((C, D2), jnp.float32) for _ in range(U)]
                    + [pltpu.VMEM_SHARED((n, D2), jnp.float32)])
        ins += [vals2, jnp.zeros((rows_last, D2), jnp.float32)]
    nin = len(ins)
    nout = len(out_type)

    @functools.partial(pl.kernel, mesh=_sc_mesh(), out_type=tuple(out_type),
                       scratch_types=tuple(scratch))
    def k(*refs):
        if vals2 is None:
            vals_hbm, idx_hbm, z_hbm, out_hbm = refs[:4]
        else:
            (vals_hbm, idx_hbm, z_hbm, v2_hbm, z2_hbm,
             out_hbm, out2_hbm) = refs[:7]
        rest = refs[nin + nout:]
        idxs = rest[:U]
        bufs = rest[U:2 * U]
        acc = rest[2 * U]
        s_i, s_v, s_a = rest[2 * U + 1:2 * U + 4]
        if vals2 is not None:
            bufs2 = rest[2 * U + 4:3 * U + 4]
            acc2 = rest[3 * U + 4]
        cid = lax.axis_index("c")
        sid = lax.axis_index("s")
        base = (sid * _NC + cid) * per_w

        def zero_wb(spm, hbm, writeback):
            if writeback:
                lo = pl.ds(sid * rows_t, rows_t)
                lo_o = pl.ds(cid * n + sid * rows_t, rows_t)
                hi = pl.ds((_NS - 1) * rows_t, rows_last)
                hi_o = pl.ds(cid * n + (_NS - 1) * rows_t, rows_last)

                @pl.when(sid < _NS - 1)
                def _():
                    pltpu.sync_copy(spm.at[lo, :], hbm.at[lo_o, :])

                @pl.when(sid == _NS - 1)
                def _():
                    pltpu.sync_copy(spm.at[hi, :], hbm.at[hi_o, :])
            else:

                @pl.when(sid < _NS - 1)
                def _():
                    pltpu.sync_copy(hbm.at[pl.ds(0, rows_t), :],
                                    spm.at[pl.ds(sid * rows_t, rows_t), :])

                @pl.when(sid == _NS - 1)
                def _():
                    pltpu.sync_copy(
                        hbm,
                        spm.at[pl.ds((_NS - 1) * rows_t, rows_last), :])

        zero_wb(acc, z_hbm, False)
        if vals2 is not None:
            zero_wb(acc2, z2_hbm, False)
        plsc.subcore_barrier()

        def step(i, carry):
            off = base + i * C * U
            cs = []
            for u in range(U):
                cs.append(pltpu.async_copy(
                    idx_hbm.at[pl.ds(off + u * C, C)], idxs[u], s_i))
                cs.append(pltpu.async_copy(
                    vals_hbm.at[pl.ds(off + u * C, C), :], bufs[u], s_v))
                if vals2 is not None:
                    cs.append(pltpu.async_copy(
                        v2_hbm.at[pl.ds(off + u * C, C), :], bufs2[u], s_v))
            adds = []
            k = 3 if vals2 is not None else 2
            for u in range(U):
                for j in range(k):
                    cs[k * u + j].wait()
                adds.append(pltpu.async_copy(
                    bufs[u], acc.at[idxs[u]], s_a, add=True))
                if vals2 is not None:
                    adds.append(pltpu.async_copy(
                        bufs2[u], acc2.at[idxs[u]], s_a, add=True))
            for a in adds:
                a.wait()
            return carry

        lax.fori_loop(0, steps, step, 0)
        plsc.subcore_barrier()
        zero_wb(acc, out_hbm, True)
        if vals2 is not None:
            zero_wb(acc2, out2_hbm, True)

    if vals2 is None:
        (out,) = k(*ins)
        return out[:n] + out[n:]
    out, out2 = k(*ins)
    return out[:n] + out[n:], out2[:n] + out2[n:]


def _pick_be(E):
    for be in (4000, 3200, 2560, 2000, 1600, 1280, 1000, 800, 640, 512, 400, 320, 256, 160, 128, 64, 32, 16, 8):
        if E % be == 0:
            return be
    return E


def _edge_body(ga_ref, gb_ref, wrad_ref, wrbf_ref, wdv_ref,
               b1_ref, we2_ref, b2_ref, watt_ref, batt_ref, *rest):
    ga = ga_ref[...]
    gb = gb_ref[...]
    fa = lax.bitcast_convert_type(ga << 16, jnp.float32)
    fb = lax.bitcast_convert_type(gb << 16, jnp.float32)
    qa = lax.bitcast_convert_type(ga & jnp.int32(-65536), jnp.float32)
    qb = lax.bitcast_convert_type(gb & jnp.int32(-65536), jnp.float32)
    xs = qa[:, 0:3] + qa[:, 6:9]
    xd = qb[:, 0:3] + qb[:, 6:9]
    cas = qa[:, 3:6] + qa[:, 9:12]
    cad = qb[:, 3:6] + qb[:, 9:12]
    cdiff = xs - xd
    dvec = cad - cas
    radial = jnp.sum(cdiff * cdiff, axis=-1, keepdims=True)
    d = jnp.sqrt(jnp.sum(dvec * dvec, axis=-1, keepdims=True) + 1e-8)
    dvn = dvec / (d + 1e-8)
    centers = (lax.broadcasted_iota(jnp.int32, (1, 16), 1).astype(jnp.float32)
               * (20.0 / 15.0))
    rbf = jnp.exp(-(((d - centers) / 1.25) ** 2))
    mpre = (fa + fb
            + radial * wrad_ref[...]
            + jnp.dot(rbf, wrbf_ref[...], preferred_element_type=jnp.float32)
            + jnp.dot(dvn, wdv_ref[...], preferred_element_type=jnp.float32)
            + b1_ref[...])
    m1 = jax.nn.silu(mpre)
    m = jax.nn.silu(
        jnp.dot(m1, we2_ref[...], preferred_element_type=jnp.float32) + b2_ref[...])
    attl = jnp.sum(m * watt_ref[...], axis=-1, keepdims=True) + batt_ref[:, :1]
    m = m * jax.nn.sigmoid(attl)
    if len(rest) == 6:
        wc1_ref, bc1_ref, wc2_ref, bc2_ref, m_ref, c8_ref = rest
        t = jax.nn.silu(
            jnp.dot(m, wc1_ref[...], preferred_element_type=jnp.float32) + bc1_ref[...])
        cw = jnp.tanh(jnp.sum(t * wc2_ref[...], axis=-1, keepdims=True) + bc2_ref[:, :1])
        cdncw = (cdiff / (jnp.sqrt(radial) + 1.0)) * cw
        m_ref[...] = m
        c8_ref[...] = jnp.concatenate(
            [cdncw, jnp.zeros_like(cdncw),
             jnp.ones((cdncw.shape[0], 1), jnp.float32),
             jnp.zeros((cdncw.shape[0], 121), jnp.float32)], axis=-1)
    else:
        (m_ref,) = rest
        m_ref[...] = m


def _edge_mlp(GA, GB, W20, b1, We2, b2, watt, batt, coords=None):
    E, W = GA.shape
    H = 128
    BE = _pick_be(E)
    grid = (E // BE,)
    row = lambda v: jnp.reshape(v, (1, -1))
    full = lambda a: pl.BlockSpec(a.shape, lambda i: (0,) * a.ndim)
    ins = [
        pl.BlockSpec((BE, W), lambda i: (i, 0)),
        pl.BlockSpec((BE, W), lambda i: (i, 0)),
    ]
    args = [GA, GB, row(W20[0]), W20[1:17], W20[17:20],
            row(b1), We2, row(b2), row(watt),
            jnp.broadcast_to(jnp.reshape(batt, (1, 1)), (1, H))]
    ins += [full(a) for a in args[2:]]
    if coords is not None:
        Wc1, bc1, wc2, bc2 = coords
        extra = [Wc1, row(bc1), row(wc2),
                 jnp.broadcast_to(jnp.reshape(bc2, (1, 1)), (1, H))]
        args += extra
        ins += [full(a) for a in extra]
        out_shape = (jax.ShapeDtypeStruct((E, H), jnp.float32),
                     jax.ShapeDtypeStruct((E, H), jnp.float32))
        out_specs = (pl.BlockSpec((BE, H), lambda i: (i, 0)),
                     pl.BlockSpec((BE, H), lambda i: (i, 0)))
    else:
        out_shape = jax.ShapeDtypeStruct((E, H), jnp.float32)
        out_specs = pl.BlockSpec((BE, H), lambda i: (i, 0))
    return pl.pallas_call(
        _edge_body,
        grid=grid,
        in_specs=ins,
        out_specs=out_specs,
        out_shape=out_shape,
    )(*args)


def kernel(X, structure_feat, seq_feat, edge_index, batch_id, params):
    p = params
    egnn = p['egnn']
    n = X.shape[0]
    src = edge_index[0]
    dst = edge_index[1]
    ca = X[:, 1, :]

    pairs = [(0, 1), (0, 2), (0, 3), (1, 2), (1, 3), (2, 3)]
    dists = jnp.stack(
        [jnp.sqrt(jnp.sum((X[:, i] - X[:, j]) ** 2, -1) + 1e-8) for i, j in pairs],
        axis=-1)
    vecs = []
    for i in (0, 2, 3):
        v = X[:, i] - ca
        vecs.append(v / (jnp.sqrt(jnp.sum(v ** 2, -1, keepdims=True)) + 1e-8))
    h_V = jnp.concatenate([dists] + vecs, axis=-1)

    sfeat = jnp.concatenate([structure_feat, h_V], axis=-1)
    h = sfeat @ egnn['emb_in'][0] + egnn['emb_in'][1]
    x = ca

    for li, lp in enumerate(egnn['layers']):
        We1w, We1b = lp['We1']
        Wsrc, Wdst = We1w[:128], We1w[128:256]
        W20 = We1w[256:]
        bf = jnp.bfloat16
        u16b = lambda v: lax.bitcast_convert_type(v.astype(bf), jnp.uint16)
        x_hi = x.astype(bf)
        x_lo = x - x_hi.astype(jnp.float32)
        ca_hi = ca.astype(bf)
        ca_lo = ca - ca_hi.astype(jnp.float32)
        Q = jnp.concatenate([x, ca, x_lo, ca_lo,
                             jnp.zeros((n, 116), jnp.float32)], axis=-1)
        qw = u16b(Q).astype(jnp.uint32) << 16
        pack = lambda P: lax.bitcast_convert_type(
            qw | u16b(P).astype(jnp.uint32), jnp.int32)
        Tsrc = pack(h @ Wsrc)
        Tdst = pack(h @ Wdst)
        GA, GB = _sc_gather_pair(Tsrc, Tdst, src, dst)
        if li == 0:
            m, c8 = _edge_mlp(
                GA, GB, W20, We1b, lp['We2'][0], lp['We2'][1],
                lp['Watt'][0][:, 0], lp['Watt'][1],
                coords=(lp['Wc1'][0], lp['Wc1'][1], lp['Wc2'][0][:, 0], lp['Wc2'][1]))
            agg = _sc_scatter_add(m, src, n)
            c8agg = _sc_scatter_add(c8, src, n)
            cnt = jnp.clip(c8agg[:, 6:7], 1.0)
            x = x + c8agg[:, :3] / cnt
        else:
            m = _edge_mlp(GA, GB, W20, We1b, lp['We2'][0], lp['We2'][1],
                          lp['Watt'][0][:, 0], lp['Watt'][1])
            agg = _sc_scatter_add(m, src, n)
        Wn1w, Wn1b = lp['Wn1']
        h_new = jax.nn.silu(h @ Wn1w[:128] + agg @ Wn1w[128:] + Wn1b)
        h = h + h_new @ lp['Wn2'][0] + lp['Wn2'][1]

    node_d1 = h @ egnn['emb_out'][0] + egnn['emb_out'][1]
    seq_d1 = seq_feat @ p['seq'][0] + p['seq'][1]
    emb = jnp.concatenate([node_d1, seq_d1], axis=-1)

    s = jnp.tanh(emb @ p['attn_fc1'][0] + p['attn_fc1'][1])
    s = s @ p['attn_fc2'][0] + p['attn_fc2'][1]
    onehot = (batch_id[:, None] == jnp.arange(_NB)[None, :]).astype(jnp.float32)
    smax = jnp.max(jnp.where(onehot[:, :, None] > 0, s[:, None, :], -jnp.inf),
                   axis=0)
    es = jnp.exp(s - onehot @ smax)
    denom = onehot.T @ es
    att = es / (onehot @ denom)
    w = jnp.sum(att, axis=-1)
    pooled = onehot.T @ (emb * w[:, None])
    emb2 = jax.nn.elu(pooled @ p['proj'][0] + p['proj'][1])
    return emb2 @ p['out'][0] + p['out'][1]


# speedup vs baseline: 5.1306x; 1.0201x over previous
import functools

import jax
import jax.numpy as jnp
from jax import lax
from jax.experimental import pallas as pl
from jax.experimental.pallas import tpu as pltpu
from jax.experimental.pallas import tpu_sc as plsc

_NB = 8
_NC = 2
_NS = 16
_NW = _NC * _NS


def _sc_mesh():
    return plsc.VectorSubcoreMesh(core_axis_name="c", subcore_axis_name="s",
                                  num_cores=_NC, num_subcores=_NS)


def _sc_gather_pair(Tsrc, Tdst, src, dst):
    n, D = Tsrc.shape
    E = src.shape[0]
    dt = Tsrc.dtype
    per_w = E // _NW
    itemsize = 2 if dt == jnp.bfloat16 else 4
    C = 80 if D * itemsize <= 512 else 40
    U = 5
    steps = per_w // (C * U)
    assert per_w == steps * C * U, (E, per_w)

    @functools.partial(
        pl.kernel,
        mesh=_sc_mesh(),
        out_type=(jax.ShapeDtypeStruct((E, D), dt),
                  jax.ShapeDtypeStruct((E, D), dt)),
        scratch_types=(
            [pltpu.VMEM((C * U,), jnp.int32) for _ in range(2)]
            + [pltpu.VMEM((C, D), dt) for _ in range(2 * U)]
            + [pltpu.SemaphoreType.DMA for _ in range(4)]),
    )
    def k(ts_hbm, td_hbm, src_hbm, dst_hbm, ga_hbm, gb_hbm,
          idxa, idxb, *rest):
        bufs = rest[:2 * U]
        s_ia, s_ib, s_g, s_w = rest[2 * U:]
        cid = lax.axis_index("c")
        sid = lax.axis_index("s")
        base = (sid * _NC + cid) * per_w

        def step(i, carry):
            off = base + i * C * U
            ca = pltpu.async_copy(src_hbm.at[pl.ds(off, C * U)], idxa, s_ia)
            cb = pltpu.async_copy(dst_hbm.at[pl.ds(off, C * U)], idxb, s_ib)
            ca.wait()
            cb.wait()
            gs = []
            for u in range(U):
                gs.append(pltpu.async_copy(
                    ts_hbm.at[idxa.at[pl.ds(u * C, C)]], bufs[2 * u], s_g))
                gs.append(pltpu.async_copy(
                    td_hbm.at[idxb.at[pl.ds(u * C, C)]], bufs[2 * u + 1], s_g))
            ws = []
            for u in range(U):
                gs[2 * u].wait()
                ws.append(pltpu.async_copy(
                    bufs[2 * u], ga_hbm.at[pl.ds(off + u * C, C), :], s_w))
                gs[2 * u + 1].wait()
                ws.append(pltpu.async_copy(
                    bufs[2 * u + 1], gb_hbm.at[pl.ds(off + u * C, C), :], s_w))
            for w in ws:
                w.wait()
            return carry

        lax.fori_loop(0, steps, step, 0)

    return k(Tsrc, Tdst, src, dst)


def _sc_scatter_add(vals, idx, n, vals2=None):
    E, D = vals.shape
    D2 = 0 if vals2 is None else vals2.shape[1]
    per_w = E // _NW
    C = 40
    U = 5
    steps = per_w // (C * U)
    assert per_w == steps * C * U, (E, per_w)
    rows_t = (n // _NS) & ~7
    rows_last = n - rows_t * (_NS - 1)
    zrows = jnp.zeros((rows_last, D), jnp.float32)

    out_type = [jax.ShapeDtypeStruct((_NC * n, D), jnp.float32)]
    scratch = ([pltpu.VMEM((C,), jnp.int32) for _ in range(U)]
               + [pltpu.VMEM((C, D), jnp.float32) for _ in range(U)]
               + [pltpu.VMEM_SHARED((n, D), jnp.float32)]
               + [pltpu.SemaphoreType.DMA for _ in range(3)])
    ins = [vals, idx, zrows]
    if vals2 is not None:
        out_type.append(jax.ShapeDtypeStruct((_NC * n, D2), jnp.float32))
        scratch += ([pltpu.VMEM((C, D2), jnp.float32) for _ in range(U)]
                    + [pltpu.VMEM_SHARED((n, D2), jnp.float32)])
        ins += [vals2, jnp.zeros((rows_last, D2), jnp.float32)]
    nin = len(ins)
    nout = len(out_type)

    @functools.partial(pl.kernel, mesh=_sc_mesh(), out_type=tuple(out_type),
                       scratch_types=tuple(scratch))
    def k(*refs):
        if vals2 is None:
            vals_hbm, idx_hbm, z_hbm, out_hbm = refs[:4]
        else:
            (vals_hbm, idx_hbm, z_hbm, v2_hbm, z2_hbm,
             out_hbm, out2_hbm) = refs[:7]
        rest = refs[nin + nout:]
        idxs = rest[:U]
        bufs = rest[U:2 * U]
        acc = rest[2 * U]
        s_i, s_v, s_a = rest[2 * U + 1:2 * U + 4]
        if vals2 is not None:
            bufs2 = rest[2 * U + 4:3 * U + 4]
            acc2 = rest[3 * U + 4]
        cid = lax.axis_index("c")
        sid = lax.axis_index("s")
        base = (sid * _NC + cid) * per_w

        def zero_wb(spm, hbm, writeback):
            if writeback:
                lo = pl.ds(sid * rows_t, rows_t)
                lo_o = pl.ds(cid * n + sid * rows_t, rows_t)
                hi = pl.ds((_NS - 1) * rows_t, rows_last)
                hi_o = pl.ds(cid * n + (_NS - 1) * rows_t, rows_last)

                @pl.when(sid < _NS - 1)
                def _():
                    pltpu.sync_copy(spm.at[lo, :], hbm.at[lo_o, :])

                @pl.when(sid == _NS - 1)
                def _():
                    pltpu.sync_copy(spm.at[hi, :], hbm.at[hi_o, :])
            else:

                @pl.when(sid < _NS - 1)
                def _():
                    pltpu.sync_copy(hbm.at[pl.ds(0, rows_t), :],
                                    spm.at[pl.ds(sid * rows_t, rows_t), :])

                @pl.when(sid == _NS - 1)
                def _():
                    pltpu.sync_copy(
                        hbm,
                        spm.at[pl.ds((_NS - 1) * rows_t, rows_last), :])

        zero_wb(acc, z_hbm, False)
        if vals2 is not None:
            zero_wb(acc2, z2_hbm, False)
        plsc.subcore_barrier()

        def step(i, carry):
            off = base + i * C * U
            cs = []
            for u in range(U):
                cs.append(pltpu.async_copy(
                    idx_hbm.at[pl.ds(off + u * C, C)], idxs[u], s_i))
                cs.append(pltpu.async_copy(
                    vals_hbm.at[pl.ds(off + u * C, C), :], bufs[u], s_v))
                if vals2 is not None:
                    cs.append(pltpu.async_copy(
                        v2_hbm.at[pl.ds(off + u * C, C), :], bufs2[u], s_v))
            adds = []
            k = 3 if vals2 is not None else 2
            for u in range(U):
                for j in range(k):
                    cs[k * u + j].wait()
                adds.append(pltpu.async_copy(
                    bufs[u], acc.at[idxs[u]], s_a, add=True))
                if vals2 is not None:
                    adds.append(pltpu.async_copy(
                        bufs2[u], acc2.at[idxs[u]], s_a, add=True))
            for a in adds:
                a.wait()
            return carry

        lax.fori_loop(0, steps, step, 0)
        plsc.subcore_barrier()
        zero_wb(acc, out_hbm, True)
        if vals2 is not None:
            zero_wb(acc2, out2_hbm, True)

    if vals2 is None:
        (out,) = k(*ins)
        return out[:n] + out[n:]
    out, out2 = k(*ins)
    return out[:n] + out[n:], out2[:n] + out2[n:]


def _pick_be(E):
    for be in (4000, 3200, 2560, 2000, 1600, 1280, 1000, 800, 640, 512, 400, 320, 256, 160, 128, 64, 32, 16, 8):
        if E % be == 0:
            return be
    return E


def _edge_body(ga_ref, gb_ref, wrad_ref, wrbf_ref, wdv_ref,
               b1_ref, we2_ref, b2_ref, watt_ref, batt_ref, *rest):
    ga = ga_ref[...]
    gb = gb_ref[...]
    fa = lax.bitcast_convert_type(ga << 16, jnp.float32)
    fb = lax.bitcast_convert_type(gb << 16, jnp.float32)
    qa = lax.bitcast_convert_type(ga[:, 0:12] & jnp.int32(-65536), jnp.float32)
    qb = lax.bitcast_convert_type(gb[:, 0:12] & jnp.int32(-65536), jnp.float32)
    xs = qa[:, 0:3] + qa[:, 6:9]
    xd = qb[:, 0:3] + qb[:, 6:9]
    cas = qa[:, 3:6] + qa[:, 9:12]
    cad = qb[:, 3:6] + qb[:, 9:12]
    cdiff = xs - xd
    dvec = cad - cas
    radial = jnp.sum(cdiff * cdiff, axis=-1, keepdims=True)
    d = jnp.sqrt(jnp.sum(dvec * dvec, axis=-1, keepdims=True) + 1e-8)
    dvn = dvec / (d + 1e-8)
    centers = (lax.broadcasted_iota(jnp.int32, (1, 16), 1).astype(jnp.float32)
               * (20.0 / 15.0))
    rbf = jnp.exp(-(((d - centers) / 1.25) ** 2))
    mpre = (fa + fb
            + radial * wrad_ref[...]
            + jnp.dot(rbf, wrbf_ref[...], preferred_element_type=jnp.float32)
            + jnp.dot(dvn, wdv_ref[...], preferred_element_type=jnp.float32)
            + b1_ref[...])
    m1 = jax.nn.silu(mpre)
    m = jax.nn.silu(
        jnp.dot(m1, we2_ref[...], preferred_element_type=jnp.float32) + b2_ref[...])
    attl = (jnp.dot(m, watt_ref[...].reshape(-1, 1),
                    preferred_element_type=jnp.float32) + batt_ref[:, :1])
    m = m * jax.nn.sigmoid(attl)
    if len(rest) == 6:
        wc1_ref, bc1_ref, wc2_ref, bc2_ref, m_ref, c8_ref = rest
        t = jax.nn.silu(
            jnp.dot(m, wc1_ref[...], preferred_element_type=jnp.float32) + bc1_ref[...])
        cw = jnp.tanh(jnp.dot(t, wc2_ref[...].reshape(-1, 1),
                              preferred_element_type=jnp.float32) + bc2_ref[:, :1])
        cdncw = (cdiff / (jnp.sqrt(radial) + 1.0)) * cw
        m_ref[...] = m
        c8_ref[...] = jnp.concatenate(
            [cdncw, jnp.zeros_like(cdncw),
             jnp.ones((cdncw.shape[0], 1), jnp.float32),
             jnp.zeros((cdncw.shape[0], 121), jnp.float32)], axis=-1)
    else:
        (m_ref,) = rest
        m_ref[...] = m


def _edge_mlp(GA, GB, W20, b1, We2, b2, watt, batt, coords=None):
    E, W = GA.shape
    H = 128
    BE = _pick_be(E)
    grid = (E // BE,)
    row = lambda v: jnp.reshape(v, (1, -1))
    full = lambda a: pl.BlockSpec(a.shape, lambda i: (0,) * a.ndim)
    ins = [
        pl.BlockSpec((BE, W), lambda i: (i, 0)),
        pl.BlockSpec((BE, W), lambda i: (i, 0)),
    ]
    args = [GA, GB, row(W20[0]), W20[1:17], W20[17:20],
            row(b1), We2, row(b2), row(watt),
            jnp.broadcast_to(jnp.reshape(batt, (1, 1)), (1, H))]
    ins += [full(a) for a in args[2:]]
    if coords is not None:
        Wc1, bc1, wc2, bc2 = coords
        extra = [Wc1, row(bc1), row(wc2),
                 jnp.broadcast_to(jnp.reshape(bc2, (1, 1)), (1, H))]
        args += extra
        ins += [full(a) for a in extra]
        out_shape = (jax.ShapeDtypeStruct((E, H), jnp.float32),
                     jax.ShapeDtypeStruct((E, H), jnp.float32))
        out_specs = (pl.BlockSpec((BE, H), lambda i: (i, 0)),
                     pl.BlockSpec((BE, H), lambda i: (i, 0)))
    else:
        out_shape = jax.ShapeDtypeStruct((E, H), jnp.float32)
        out_specs = pl.BlockSpec((BE, H), lambda i: (i, 0))
    return pl.pallas_call(
        _edge_body,
        grid=grid,
        in_specs=ins,
        out_specs=out_specs,
        out_shape=out_shape,
    )(*args)


def kernel(X, structure_feat, seq_feat, edge_index, batch_id, params):
    p = params
    egnn = p['egnn']
    n = X.shape[0]
    src = edge_index[0]
    dst = edge_index[1]
    ca = X[:, 1, :]

    pairs = [(0, 1), (0, 2), (0, 3), (1, 2), (1, 3), (2, 3)]
    dists = jnp.stack(
        [jnp.sqrt(jnp.sum((X[:, i] - X[:, j]) ** 2, -1) + 1e-8) for i, j in pairs],
        axis=-1)
    vecs = []
    for i in (0, 2, 3):
        v = X[:, i] - ca
        vecs.append(v / (jnp.sqrt(jnp.sum(v ** 2, -1, keepdims=True)) + 1e-8))
    h_V = jnp.concatenate([dists] + vecs, axis=-1)

    sfeat = jnp.concatenate([structure_feat, h_V], axis=-1)
    h = sfeat @ egnn['emb_in'][0] + egnn['emb_in'][1]
    x = ca

    for li, lp in enumerate(egnn['layers']):
        We1w, We1b = lp['We1']
        Wsrc, Wdst = We1w[:128], We1w[128:256]
        W20 = We1w[256:]
        bf = jnp.bfloat16
        u16b = lambda v: lax.bitcast_convert_type(v.astype(bf), jnp.uint16)
        x_hi = x.astype(bf)
        x_lo = x - x_hi.astype(jnp.float32)
        ca_hi = ca.astype(bf)
        ca_lo = ca - ca_hi.astype(jnp.float32)
        Q = jnp.concatenate([x, ca, x_lo, ca_lo,
                             jnp.zeros((n, 116), jnp.float32)], axis=-1)
        qw = u16b(Q).astype(jnp.uint32) << 16
        pack = lambda P: lax.bitcast_convert_type(
            qw | u16b(P).astype(jnp.uint32), jnp.int32)
        Tsrc = pack(h @ Wsrc)
        Tdst = pack(h @ Wdst)
        GA, GB = _sc_gather_pair(Tsrc, Tdst, src, dst)
        if li == 0:
            m, c8 = _edge_mlp(
                GA, GB, W20, We1b, lp['We2'][0], lp['We2'][1],
                lp['Watt'][0][:, 0], lp['Watt'][1],
                coords=(lp['Wc1'][0], lp['Wc1'][1], lp['Wc2'][0][:, 0], lp['Wc2'][1]))
            agg = _sc_scatter_add(m, src, n)
            c8agg = _sc_scatter_add(c8, src, n)
            cnt = jnp.clip(c8agg[:, 6:7], 1.0)
            x = x + c8agg[:, :3] / cnt
        else:
            m = _edge_mlp(GA, GB, W20, We1b, lp['We2'][0], lp['We2'][1],
                          lp['Watt'][0][:, 0], lp['Watt'][1])
            agg = _sc_scatter_add(m, src, n)
        Wn1w, Wn1b = lp['Wn1']
        h_new = jax.nn.silu(h @ Wn1w[:128] + agg @ Wn1w[128:] + Wn1b)
        h = h + h_new @ lp['Wn2'][0] + lp['Wn2'][1]

    node_d1 = h @ egnn['emb_out'][0] + egnn['emb_out'][1]
    seq_d1 = seq_feat @ p['seq'][0] + p['seq'][1]
    emb = jnp.concatenate([node_d1, seq_d1], axis=-1)

    s = jnp.tanh(emb @ p['attn_fc1'][0] + p['attn_fc1'][1])
    s = s @ p['attn_fc2'][0] + p['attn_fc2'][1]
    onehot = (batch_id[:, None] == jnp.arange(_NB)[None, :]).astype(jnp.float32)
    smax = jnp.max(jnp.where(onehot[:, :, None] > 0, s[:, None, :], -jnp.inf),
                   axis=0)
    es = jnp.exp(s - onehot @ smax)
    denom = onehot.T @ es
    att = es / (onehot @ denom)
    w = jnp.sum(att, axis=-1)
    pooled = onehot.T @ (emb * w[:, None])
    emb2 = jax.nn.elu(pooled @ p['proj'][0] + p['proj'][1])
    return emb2 @ p['out'][0] + p['out'][1]


# speedup vs baseline: 5.1359x; 1.0010x over previous
import functools

import jax
import jax.numpy as jnp
from jax import lax
from jax.experimental import pallas as pl
from jax.experimental.pallas import tpu as pltpu
from jax.experimental.pallas import tpu_sc as plsc

_NB = 8
_NC = 2
_NS = 16
_NW = _NC * _NS


def _sc_mesh():
    return plsc.VectorSubcoreMesh(core_axis_name="c", subcore_axis_name="s",
                                  num_cores=_NC, num_subcores=_NS)


def _sc_gather_pair(Tsrc, Tdst, src, dst):
    n, D = Tsrc.shape
    E = src.shape[0]
    dt = Tsrc.dtype
    per_w = E // _NW
    itemsize = 2 if dt == jnp.bfloat16 else 4
    C = 80 if D * itemsize <= 512 else 40
    U = 5
    steps = per_w // (C * U)
    assert per_w == steps * C * U, (E, per_w)

    @functools.partial(
        pl.kernel,
        mesh=_sc_mesh(),
        out_type=(jax.ShapeDtypeStruct((E, D), dt),
                  jax.ShapeDtypeStruct((E, D), dt)),
        scratch_types=(
            [pltpu.VMEM((C * U,), jnp.int32) for _ in range(2)]
            + [pltpu.VMEM((C, D), dt) for _ in range(2 * U)]
            + [pltpu.SemaphoreType.DMA for _ in range(4)]),
    )
    def k(ts_hbm, td_hbm, src_hbm, dst_hbm, ga_hbm, gb_hbm,
          idxa, idxb, *rest):
        bufs = rest[:2 * U]
        s_ia, s_ib, s_g, s_w = rest[2 * U:]
        cid = lax.axis_index("c")
        sid = lax.axis_index("s")
        base = (sid * _NC + cid) * per_w

        def step(i, carry):
            off = base + i * C * U
            ca = pltpu.async_copy(src_hbm.at[pl.ds(off, C * U)], idxa, s_ia)
            cb = pltpu.async_copy(dst_hbm.at[pl.ds(off, C * U)], idxb, s_ib)
            ca.wait()
            cb.wait()
            gs = []
            for u in range(U):
                gs.append(pltpu.async_copy(
                    ts_hbm.at[idxa.at[pl.ds(u * C, C)]], bufs[2 * u], s_g))
                gs.append(pltpu.async_copy(
                    td_hbm.at[idxb.at[pl.ds(u * C, C)]], bufs[2 * u + 1], s_g))
            ws = []
            for u in range(U):
                gs[2 * u].wait()
                ws.append(pltpu.async_copy(
                    bufs[2 * u], ga_hbm.at[pl.ds(off + u * C, C), :], s_w))
                gs[2 * u + 1].wait()
                ws.append(pltpu.async_copy(
                    bufs[2 * u + 1], gb_hbm.at[pl.ds(off + u * C, C), :], s_w))
            for w in ws:
                w.wait()
            return carry

        lax.fori_loop(0, steps, step, 0)

    return k(Tsrc, Tdst, src, dst)


def _sc_scatter_add(vals, idx, n, vals2=None, lanes=None):
    E, D = vals.shape
    if lanes is not None:
        D = lanes
    D2 = 0 if vals2 is None else vals2.shape[1]
    per_w = E // _NW
    C = 40
    U = 5
    steps = per_w // (C * U)
    assert per_w == steps * C * U, (E, per_w)
    rows_t = (n // _NS) & ~7
    rows_last = n - rows_t * (_NS - 1)
    zrows = jnp.zeros((rows_last, D), jnp.float32)

    out_type = [jax.ShapeDtypeStruct((_NC * n, D), jnp.float32)]
    scratch = ([pltpu.VMEM((C,), jnp.int32) for _ in range(U)]
               + [pltpu.VMEM((C, D), jnp.float32) for _ in range(U)]
               + [pltpu.VMEM_SHARED((n, D), jnp.float32)]
               + [pltpu.SemaphoreType.DMA for _ in range(3)])
    ins = [vals, idx, zrows]
    if vals2 is not None:
        out_type.append(jax.ShapeDtypeStruct((_NC * n, D2), jnp.float32))
        scratch += ([pltpu.VMEM((C, D2), jnp.float32) for _ in range(U)]
                    + [pltpu.VMEM_SHARED((n, D2), jnp.float32)])
        ins += [vals2, jnp.zeros((rows_last, D2), jnp.float32)]
    nin = len(ins)
    nout = len(out_type)

    @functools.partial(pl.kernel, mesh=_sc_mesh(), out_type=tuple(out_type),
                       scratch_types=tuple(scratch))
    def k(*refs):
        if vals2 is None:
            vals_hbm, idx_hbm, z_hbm, out_hbm = refs[:4]
        else:
            (vals_hbm, idx_hbm, z_hbm, v2_hbm, z2_hbm,
             out_hbm, out2_hbm) = refs[:7]
        rest = refs[nin + nout:]
        idxs = rest[:U]
        bufs = rest[U:2 * U]
        acc = rest[2 * U]
        s_i, s_v, s_a = rest[2 * U + 1:2 * U + 4]
        if vals2 is not None:
            bufs2 = rest[2 * U + 4:3 * U + 4]
            acc2 = rest[3 * U + 4]
        cid = lax.axis_index("c")
        sid = lax.axis_index("s")
        base = (sid * _NC + cid) * per_w

        def zero_wb(spm, hbm, writeback):
            if writeback:
                lo = pl.ds(sid * rows_t, rows_t)
                lo_o = pl.ds(cid * n + sid * rows_t, rows_t)
                hi = pl.ds((_NS - 1) * rows_t, rows_last)
                hi_o = pl.ds(cid * n + (_NS - 1) * rows_t, rows_last)

                @pl.when(sid < _NS - 1)
                def _():
                    pltpu.sync_copy(spm.at[lo, :], hbm.at[lo_o, :])

                @pl.when(sid == _NS - 1)
                def _():
                    pltpu.sync_copy(spm.at[hi, :], hbm.at[hi_o, :])
            else:

                @pl.when(sid < _NS - 1)
                def _():
                    pltpu.sync_copy(hbm.at[pl.ds(0, rows_t), :],
                                    spm.at[pl.ds(sid * rows_t, rows_t), :])

                @pl.when(sid == _NS - 1)
                def _():
                    pltpu.sync_copy(
                        hbm,
                        spm.at[pl.ds((_NS - 1) * rows_t, rows_last), :])

        zero_wb(acc, z_hbm, False)
        if vals2 is not None:
            zero_wb(acc2, z2_hbm, False)
        plsc.subcore_barrier()

        def step(i, carry):
            off = base + i * C * U
            cs = []
            for u in range(U):
                cs.append(pltpu.async_copy(
                    idx_hbm.at[pl.ds(off + u * C, C)], idxs[u], s_i))
                cs.append(pltpu.async_copy(
                    vals_hbm.at[pl.ds(off + u * C, C), pl.ds(0, D)], bufs[u], s_v))
                if vals2 is not None:
                    cs.append(pltpu.async_copy(
                        v2_hbm.at[pl.ds(off + u * C, C), :], bufs2[u], s_v))
            adds = []
            k = 3 if vals2 is not None else 2
            for u in range(U):
                for j in range(k):
                    cs[k * u + j].wait()
                adds.append(pltpu.async_copy(
                    bufs[u], acc.at[idxs[u]], s_a, add=True))
                if vals2 is not None:
                    adds.append(pltpu.async_copy(
                        bufs2[u], acc2.at[idxs[u]], s_a, add=True))
            for a in adds:
                a.wait()
            return carry

        lax.fori_loop(0, steps, step, 0)
        plsc.subcore_barrier()
        zero_wb(acc, out_hbm, True)
        if vals2 is not None:
            zero_wb(acc2, out2_hbm, True)

    if vals2 is None:
        (out,) = k(*ins)
        return out[:n] + out[n:]
    out, out2 = k(*ins)
    return out[:n] + out[n:], out2[:n] + out2[n:]


def _pick_be(E):
    for be in (4000, 3200, 2560, 2000, 1600, 1280, 1000, 800, 640, 512, 400, 320, 256, 160, 128, 64, 32, 16, 8):
        if E % be == 0:
            return be
    return E


def _edge_body(ga_ref, gb_ref, wrad_ref, wrbf_ref, wdv_ref,
               b1_ref, we2_ref, b2_ref, watt_ref, batt_ref, *rest):
    ga = ga_ref[...]
    gb = gb_ref[...]
    fa = lax.bitcast_convert_type(ga << 16, jnp.float32)
    fb = lax.bitcast_convert_type(gb << 16, jnp.float32)
    qa = lax.bitcast_convert_type(ga[:, 0:12] & jnp.int32(-65536), jnp.float32)
    qb = lax.bitcast_convert_type(gb[:, 0:12] & jnp.int32(-65536), jnp.float32)
    xs = qa[:, 0:3] + qa[:, 6:9]
    xd = qb[:, 0:3] + qb[:, 6:9]
    cas = qa[:, 3:6] + qa[:, 9:12]
    cad = qb[:, 3:6] + qb[:, 9:12]
    cdiff = xs - xd
    dvec = cad - cas
    radial = jnp.sum(cdiff * cdiff, axis=-1, keepdims=True)
    d = jnp.sqrt(jnp.sum(dvec * dvec, axis=-1, keepdims=True) + 1e-8)
    dvn = dvec / (d + 1e-8)
    centers = (lax.broadcasted_iota(jnp.int32, (1, 16), 1).astype(jnp.float32)
               * (20.0 / 15.0))
    rbf = jnp.exp(-(((d - centers) / 1.25) ** 2))
    mpre = (fa + fb
            + radial * wrad_ref[...]
            + jnp.dot(rbf, wrbf_ref[...], preferred_element_type=jnp.float32)
            + jnp.dot(dvn, wdv_ref[...], preferred_element_type=jnp.float32)
            + b1_ref[...])
    m1 = jax.nn.silu(mpre)
    m = jax.nn.silu(
        jnp.dot(m1, we2_ref[...], preferred_element_type=jnp.float32) + b2_ref[...])
    attl = (jnp.dot(m, watt_ref[...].reshape(-1, 1),
                    preferred_element_type=jnp.float32) + batt_ref[:, :1])
    m = m * jax.nn.sigmoid(attl)
    if len(rest) == 6:
        wc1_ref, bc1_ref, wc2_ref, bc2_ref, m_ref, c8_ref = rest
        t = jax.nn.silu(
            jnp.dot(m, wc1_ref[...], preferred_element_type=jnp.float32) + bc1_ref[...])
        cw = jnp.tanh(jnp.dot(t, wc2_ref[...].reshape(-1, 1),
                              preferred_element_type=jnp.float32) + bc2_ref[:, :1])
        cdncw = (cdiff / (jnp.sqrt(radial) + 1.0)) * cw
        m_ref[...] = m
        c8_ref[...] = jnp.concatenate(
            [cdncw, jnp.zeros_like(cdncw),
             jnp.ones((cdncw.shape[0], 1), jnp.float32),
             jnp.zeros((cdncw.shape[0], 121), jnp.float32)], axis=-1)
    else:
        (m_ref,) = rest
        m_ref[...] = m


def _edge_mlp(GA, GB, W20, b1, We2, b2, watt, batt, coords=None):
    E, W = GA.shape
    H = 128
    BE = _pick_be(E)
    grid = (E // BE,)
    row = lambda v: jnp.reshape(v, (1, -1))
    full = lambda a: pl.BlockSpec(a.shape, lambda i: (0,) * a.ndim)
    ins = [
        pl.BlockSpec((BE, W), lambda i: (i, 0)),
        pl.BlockSpec((BE, W), lambda i: (i, 0)),
    ]
    args = [GA, GB, row(W20[0]), W20[1:17], W20[17:20],
            row(b1), We2, row(b2), row(watt),
            jnp.broadcast_to(jnp.reshape(batt, (1, 1)), (1, H))]
    ins += [full(a) for a in args[2:]]
    if coords is not None:
        Wc1, bc1, wc2, bc2 = coords
        extra = [Wc1, row(bc1), row(wc2),
                 jnp.broadcast_to(jnp.reshape(bc2, (1, 1)), (1, H))]
        args += extra
        ins += [full(a) for a in extra]
        out_shape = (jax.ShapeDtypeStruct((E, H), jnp.float32),
                     jax.ShapeDtypeStruct((E, H), jnp.float32))
        out_specs = (pl.BlockSpec((BE, H), lambda i: (i, 0)),
                     pl.BlockSpec((BE, H), lambda i: (i, 0)))
    else:
        out_shape = jax.ShapeDtypeStruct((E, H), jnp.float32)
        out_specs = pl.BlockSpec((BE, H), lambda i: (i, 0))
    return pl.pallas_call(
        _edge_body,
        grid=grid,
        in_specs=ins,
        out_specs=out_specs,
        out_shape=out_shape,
    )(*args)


def kernel(X, structure_feat, seq_feat, edge_index, batch_id, params):
    p = params
    egnn = p['egnn']
    n = X.shape[0]
    src = edge_index[0]
    dst = edge_index[1]
    ca = X[:, 1, :]

    pairs = [(0, 1), (0, 2), (0, 3), (1, 2), (1, 3), (2, 3)]
    dists = jnp.stack(
        [jnp.sqrt(jnp.sum((X[:, i] - X[:, j]) ** 2, -1) + 1e-8) for i, j in pairs],
        axis=-1)
    vecs = []
    for i in (0, 2, 3):
        v = X[:, i] - ca
        vecs.append(v / (jnp.sqrt(jnp.sum(v ** 2, -1, keepdims=True)) + 1e-8))
    h_V = jnp.concatenate([dists] + vecs, axis=-1)

    sfeat = jnp.concatenate([structure_feat, h_V], axis=-1)
    h = sfeat @ egnn['emb_in'][0] + egnn['emb_in'][1]
    x = ca

    for li, lp in enumerate(egnn['layers']):
        We1w, We1b = lp['We1']
        Wsrc, Wdst = We1w[:128], We1w[128:256]
        W20 = We1w[256:]
        bf = jnp.bfloat16
        u16b = lambda v: lax.bitcast_convert_type(v.astype(bf), jnp.uint16)
        x_hi = x.astype(bf)
        x_lo = x - x_hi.astype(jnp.float32)
        ca_hi = ca.astype(bf)
        ca_lo = ca - ca_hi.astype(jnp.float32)
        Q = jnp.concatenate([x, ca, x_lo, ca_lo,
                             jnp.zeros((n, 116), jnp.float32)], axis=-1)
        qw = u16b(Q).astype(jnp.uint32) << 16
        pack = lambda P: lax.bitcast_convert_type(
            qw | u16b(P).astype(jnp.uint32), jnp.int32)
        Tsrc = pack(h @ Wsrc)
        Tdst = pack(h @ Wdst)
        GA, GB = _sc_gather_pair(Tsrc, Tdst, src, dst)
        if li == 0:
            m, c8 = _edge_mlp(
                GA, GB, W20, We1b, lp['We2'][0], lp['We2'][1],
                lp['Watt'][0][:, 0], lp['Watt'][1],
                coords=(lp['Wc1'][0], lp['Wc1'][1], lp['Wc2'][0][:, 0], lp['Wc2'][1]))
            agg = _sc_scatter_add(m, src, n)
            c8agg = _sc_scatter_add(c8, src, n)
            cnt = jnp.clip(c8agg[:, 6:7], 1.0)
            x = x + c8agg[:, :3] / cnt
        else:
            m = _edge_mlp(GA, GB, W20, We1b, lp['We2'][0], lp['We2'][1],
                          lp['Watt'][0][:, 0], lp['Watt'][1])
            agg = _sc_scatter_add(m, src, n)
        Wn1w, Wn1b = lp['Wn1']
        h_new = jax.nn.silu(h @ Wn1w[:128] + agg @ Wn1w[128:] + Wn1b)
        h = h + h_new @ lp['Wn2'][0] + lp['Wn2'][1]

    node_d1 = h @ egnn['emb_out'][0] + egnn['emb_out'][1]
    seq_d1 = seq_feat @ p['seq'][0] + p['seq'][1]
    emb = jnp.concatenate([node_d1, seq_d1], axis=-1)

    s = jnp.tanh(emb @ p['attn_fc1'][0] + p['attn_fc1'][1])
    s = s @ p['attn_fc2'][0] + p['attn_fc2'][1]
    onehot = (batch_id[:, None] == jnp.arange(_NB)[None, :]).astype(jnp.float32)
    smax = jnp.max(jnp.where(onehot[:, :, None] > 0, s[:, None, :], -jnp.inf),
                   axis=0)
    es = jnp.exp(s - onehot @ smax)
    denom = onehot.T @ es
    att = es / (onehot @ denom)
    w = jnp.sum(att, axis=-1)
    pooled = onehot.T @ (emb * w[:, None])
    emb2 = jax.nn.elu(pooled @ p['proj'][0] + p['proj'][1])
    return emb2 @ p['out'][0] + p['out'][1]
